# Initial kernel scaffold; baseline (speedup 1.0000x reference)
import jax
import jax.numpy as jnp
from jax.experimental import pallas as pl


def kernel(user_x, food_x, edge_index, W_user, b_user, W_food, b_food, W1, b1, W2, b2):
    raise NotImplementedError("write your pallas kernel here")


# speedup vs baseline: 17.4994x; 17.4994x over previous
import functools

import jax
import jax.numpy as jnp
from jax import lax
from jax.experimental import pallas as pl
from jax.experimental.pallas import tpu as pltpu
from jax.experimental.pallas import tpu_sc as plsc

N_USERS = 2000
N_ITEMS = 8000
N_NODES = N_USERS + N_ITEMS
D = 128
DH = D // 2
E = 320000

NC = 2
NS = 16
NW = NC * NS
CHUNK = 128
NCHUNK_DEG = 80
NCHUNK = 160
E_PAD = NW * NCHUNK_DEG * CHUNK
N_PAD = 10240
RPT = N_PAD // NS

_MESH = dict(core_axis_name="c", subcore_axis_name="s")


def _deg_body(dstr_hbm, zeros1_hbm, out_hbm, dst_v, ones_v, acc_sh):
    c = lax.axis_index("c")
    s = lax.axis_index("s")
    w = c * NS + s
    pltpu.sync_copy(zeros1_hbm.at[pl.ds(s * RPT, RPT)],
                    acc_sh.at[pl.ds(s * RPT, RPT)])
    pltpu.sync_copy(dstr_hbm.at[w], dst_v)
    for j in range(CHUNK // 16):
        ones_v[pl.ds(j * 16, 16)] = jnp.full((16,), 1.0, dtype=jnp.float32)
    plsc.subcore_barrier()

    def body(i, carry):
        pltpu.sync_copy(ones_v, acc_sh.at[dst_v.at[i]], add=True)
        return carry

    lax.fori_loop(0, NCHUNK_DEG, body, 0)
    plsc.subcore_barrier()
    pltpu.sync_copy(acc_sh.at[pl.ds(s * RPT, RPT)],
                    out_hbm.at[c, pl.ds(s * RPT, RPT)])


@functools.cache
def _deg_call():
    return pl.kernel(
        _deg_body,
        out_type=jax.ShapeDtypeStruct((NC, N_PAD), jnp.float32),
        mesh=plsc.VectorSubcoreMesh(**_MESH),
        scratch_types=[
            pltpu.VMEM((NCHUNK_DEG, CHUNK), jnp.int32),
            pltpu.VMEM((CHUNK,), jnp.float32),
            pltpu.VMEM_SHARED((N_PAD,), jnp.float32),
        ],
    )


def _scat_body(table_hbm, srcr_hbm, dstr_hbm, zeros2_hbm, out_hbm,
               src_v, dst_v, rows0, rows1, acc_sh, sem0, sem1):
    c = lax.axis_index("c")
    s = lax.axis_index("s")
    half = table_hbm.at[c]
    pltpu.sync_copy(zeros2_hbm.at[pl.ds(s * RPT, RPT)],
                    acc_sh.at[pl.ds(s * RPT, RPT)])
    pltpu.sync_copy(srcr_hbm.at[s], src_v)
    pltpu.sync_copy(dstr_hbm.at[s], dst_v)
    plsc.subcore_barrier()

    bufs = (rows0, rows1)
    sems = (sem0, sem1)

    def start(i, b):
        pltpu.async_copy(half.at[src_v.at[i]], bufs[b], sems[b])

    def wait(i, b):
        pltpu.make_async_copy(half.at[src_v.at[i]], bufs[b], sems[b]).wait()

    def scat(i, b):
        pltpu.sync_copy(bufs[b], acc_sh.at[dst_v.at[i]], add=True)

    start(0, 0)
    start(1, 1)

    def body(it, carry):
        i0 = it * 2
        for b in (0, 1):
            wait(i0 + b, b)
            scat(i0 + b, b)
            start(i0 + b + 2, b)
        return carry

    lax.fori_loop(0, (NCHUNK - 2) // 2, body, 0)
    for b in (0, 1):
        wait(NCHUNK - 2 + b, b)
        scat(NCHUNK - 2 + b, b)

    plsc.subcore_barrier()
    pltpu.sync_copy(acc_sh.at[pl.ds(s * RPT, RPT)],
                    out_hbm.at[c, pl.ds(s * RPT, RPT)])


@functools.cache
def _scat_call():
    return pl.kernel(
        _scat_body,
        out_type=jax.ShapeDtypeStruct((NC, N_PAD, DH), jnp.float32),
        mesh=plsc.VectorSubcoreMesh(**_MESH),
        scratch_types=[
            pltpu.VMEM((NCHUNK, CHUNK), jnp.int32),
            pltpu.VMEM((NCHUNK, CHUNK), jnp.int32),
            pltpu.VMEM((CHUNK, DH), jnp.float32),
            pltpu.VMEM((CHUNK, DH), jnp.float32),
            pltpu.VMEM_SHARED((N_PAD, DH), jnp.float32),
            pltpu.SemaphoreType.DMA,
            pltpu.SemaphoreType.DMA,
        ],
        compiler_params=pltpu.CompilerParams(use_tc_tiling_on_sc=False),
    )


_R = 256
_GRID = N_PAD // _R


def _dinv_of(degp_ref):
    deg = degp_ref[0] + degp_ref[1]
    return jnp.where(deg > 0.0, lax.rsqrt(jnp.maximum(deg, 1.0)), 0.0)


def _split_store(out_ref, m):
    out_ref[0] = m[:, :DH]
    out_ref[1] = m[:, DH:]


def _proj_body(x_ref, degp_ref, wu_ref, bu_ref, wf_ref, bf_ref, w1_ref,
               out_ref):
    i = pl.program_id(0)
    x = x_ref[...]
    hu = jnp.maximum(jnp.dot(x, wu_ref[...],
                             preferred_element_type=jnp.float32)
                     + bu_ref[...], 0.0)
    hf = jnp.maximum(jnp.dot(x, wf_ref[...],
                             preferred_element_type=jnp.float32)
                     + bf_ref[...], 0.0)
    row = i * _R + lax.broadcasted_iota(jnp.int32, (_R, 1), 0)
    h = jnp.where(row < N_USERS, hu, hf)
    m = jnp.dot(h, w1_ref[...], preferred_element_type=jnp.float32)
    _split_store(out_ref, _dinv_of(degp_ref) * m)


def _proj_call(x, degp3, wu, bu, wf, bf, w1):
    return pl.pallas_call(
        _proj_body,
        grid=(_GRID,),
        in_specs=[
            pl.BlockSpec((_R, D), lambda i: (i, 0)),
            pl.BlockSpec((NC, _R, 1), lambda i: (0, i, 0)),
            pl.BlockSpec((D, D), lambda i: (0, 0)),
            pl.BlockSpec((1, D), lambda i: (0, 0)),
            pl.BlockSpec((D, D), lambda i: (0, 0)),
            pl.BlockSpec((1, D), lambda i: (0, 0)),
            pl.BlockSpec((D, D), lambda i: (0, 0)),
        ],
        out_specs=pl.BlockSpec((NC, _R, DH), lambda i: (0, i, 0)),
        out_shape=jax.ShapeDtypeStruct((NC, N_PAD, DH), jnp.float32),
    )(x, degp3, wu, bu, wf, bf, w1)


def _mid_body(acc_ref, degp_ref, b_ref, w_ref, out_ref):
    dinv = _dinv_of(degp_ref)
    agg = jnp.concatenate([acc_ref[0], acc_ref[1]], axis=1)
    h = jnp.maximum(dinv * agg + b_ref[...], 0.0)
    m = jnp.dot(h, w_ref[...], preferred_element_type=jnp.float32)
    _split_store(out_ref, dinv * m)


def _mid_call(acc, degp3, b, w):
    return pl.pallas_call(
        _mid_body,
        grid=(_GRID,),
        in_specs=[
            pl.BlockSpec((NC, _R, DH), lambda i: (0, i, 0)),
            pl.BlockSpec((NC, _R, 1), lambda i: (0, i, 0)),
            pl.BlockSpec((1, D), lambda i: (0, 0)),
            pl.BlockSpec((D, D), lambda i: (0, 0)),
        ],
        out_specs=pl.BlockSpec((NC, _R, DH), lambda i: (0, i, 0)),
        out_shape=jax.ShapeDtypeStruct((NC, N_PAD, DH), jnp.float32),
    )(acc, degp3, b, w)


def _fin_body(acc_ref, degp_ref, b_ref, out_ref):
    dinv = _dinv_of(degp_ref)
    agg = jnp.concatenate([acc_ref[0], acc_ref[1]], axis=1)
    out_ref[...] = jnp.maximum(dinv * agg + b_ref[...], 0.0)


def _fin_call(acc, degp3, b):
    return pl.pallas_call(
        _fin_body,
        grid=(_GRID,),
        in_specs=[
            pl.BlockSpec((NC, _R, DH), lambda i: (0, i, 0)),
            pl.BlockSpec((NC, _R, 1), lambda i: (0, i, 0)),
            pl.BlockSpec((1, D), lambda i: (0, 0)),
        ],
        out_specs=pl.BlockSpec((_R, D), lambda i: (i, 0)),
        out_shape=jax.ShapeDtypeStruct((N_PAD, D), jnp.float32),
    )(acc, degp3, b)


def kernel(user_x, food_x, edge_index, W_user, b_user, W_food, b_food,
           W1, b1, W2, b2):
    src = edge_index[0]
    dst = edge_index[1]
    pad_idx = N_NODES + (jnp.arange(E_PAD - E, dtype=jnp.int32)
                         % (N_PAD - N_NODES))
    src_p = jnp.concatenate([src, pad_idx])
    dst_p = jnp.concatenate([dst, pad_idx])
    src_r = src_p.reshape(NS, NCHUNK, CHUNK)
    dst_r = dst_p.reshape(NS, NCHUNK, CHUNK)
    dst_r_deg = dst_p.reshape(NW, NCHUNK_DEG, CHUNK)
    x_pad = jnp.concatenate(
        [user_x, food_x, jnp.zeros((N_PAD - N_NODES, D), jnp.float32)], axis=0)
    zeros1 = jnp.zeros((N_PAD,), jnp.float32)
    zeros2 = jnp.zeros((N_PAD, DH), jnp.float32)

    degp = _deg_call()(dst_r_deg, zeros1)
    degp3 = degp.reshape(NC, N_PAD, 1)

    m1 = _proj_call(x_pad, degp3, W_user, b_user.reshape(1, D),
                    W_food, b_food.reshape(1, D), W1)
    acc1 = _scat_call()(m1, src_r, dst_r, zeros2)
    m2 = _mid_call(acc1, degp3, b1.reshape(1, D), W2)
    acc2 = _scat_call()(m2, src_r, dst_r, zeros2)
    h2 = _fin_call(acc2, degp3, b2.reshape(1, D))

    users = h2[:N_USERS]
    items = h2[N_USERS:N_NODES]
    return (users, users, items, items)


# speedup vs baseline: 23.4470x; 1.3399x over previous
import functools

import jax
import jax.numpy as jnp
from jax import lax
from jax.experimental import pallas as pl
from jax.experimental.pallas import tpu as pltpu
from jax.experimental.pallas import tpu_sc as plsc

N_USERS = 2000
N_ITEMS = 8000
N_NODES = N_USERS + N_ITEMS
D = 128
E = 320000

NC = 2
NS = 16
NW = NC * NS
CHUNK = 128
NCHUNK = 80
E_PAD = NW * NCHUNK * CHUNK
N_PAD = 10240
RPT = N_PAD // NS

_MESH = dict(core_axis_name="c", subcore_axis_name="s")


def _deg_body(dstr_hbm, zeros1_hbm, out_hbm, dst_v, ones_v, acc_sh):
    c = lax.axis_index("c")
    s = lax.axis_index("s")
    w = c * NS + s
    pltpu.sync_copy(zeros1_hbm.at[pl.ds(s * RPT, RPT)],
                    acc_sh.at[pl.ds(s * RPT, RPT)])
    pltpu.sync_copy(dstr_hbm.at[w], dst_v)
    for j in range(CHUNK // 16):
        ones_v[pl.ds(j * 16, 16)] = jnp.full((16,), 1.0, dtype=jnp.float32)
    plsc.subcore_barrier()

    def body(i, carry):
        pltpu.sync_copy(ones_v, acc_sh.at[dst_v.at[i]], add=True)
        return carry

    lax.fori_loop(0, NCHUNK, body, 0)
    plsc.subcore_barrier()
    pltpu.sync_copy(acc_sh.at[pl.ds(s * RPT, RPT)],
                    out_hbm.at[c, pl.ds(s * RPT, RPT)])


@functools.cache
def _deg_call():
    return pl.kernel(
        _deg_body,
        out_type=jax.ShapeDtypeStruct((NC, N_PAD), jnp.float32),
        mesh=plsc.VectorSubcoreMesh(**_MESH),
        scratch_types=[
            pltpu.VMEM((NCHUNK, CHUNK), jnp.int32),
            pltpu.VMEM((CHUNK,), jnp.float32),
            pltpu.VMEM_SHARED((N_PAD,), jnp.float32),
        ],
    )


GROUP = 16
NGROUP = NCHUNK // GROUP


def _scat_body(table_hbm, srcr_hbm, dstr_hbm, zeros2_hbm, out_hbm,
               srcs0, srcs1, dsts0, dsts1, rows0, rows1, acc_sh,
               i0sem, i1sem, g0, g1, s0, s1):
    c = lax.axis_index("c")
    s = lax.axis_index("s")
    w = c * NS + s

    src_s = (srcs0, srcs1)
    dst_s = (dsts0, dsts1)
    isem = (i0sem, i1sem)
    bufs = (rows0, rows1)
    gsem = (g0, g1)
    ssem = (s0, s1)

    def idx_start(g, slot):
        pltpu.async_copy(srcr_hbm.at[w, pl.ds(g * GROUP, GROUP)],
                         src_s[slot], isem[slot])
        pltpu.async_copy(dstr_hbm.at[w, pl.ds(g * GROUP, GROUP)],
                         dst_s[slot], isem[slot])

    def idx_wait(g, slot):
        pltpu.make_async_copy(srcr_hbm.at[w, pl.ds(g * GROUP, GROUP)],
                              src_s[slot], isem[slot]).wait()
        pltpu.make_async_copy(dstr_hbm.at[w, pl.ds(g * GROUP, GROUP)],
                              dst_s[slot], isem[slot]).wait()

    def g_start(slot, j, b):
        pltpu.async_copy(table_hbm.at[src_s[slot].at[j]], bufs[b], gsem[b])

    def g_wait(slot, j, b):
        pltpu.make_async_copy(table_hbm.at[src_s[slot].at[j]], bufs[b],
                              gsem[b]).wait()

    def s_start(slot, j, b):
        pltpu.async_copy(bufs[b], acc_sh.at[dst_s[slot].at[j]], ssem[b],
                         add=True)

    def s_wait(slot, j, b):
        pltpu.make_async_copy(bufs[b], acc_sh.at[dst_s[slot].at[j]],
                              ssem[b]).wait()

    idx_start(0, 0)
    pltpu.sync_copy(zeros2_hbm.at[pl.ds(s * RPT, RPT)],
                    acc_sh.at[pl.ds(s * RPT, RPT)])
    plsc.subcore_barrier()

    for g in range(NGROUP):
        slot = g % 2
        idx_wait(g, slot)
        if g + 1 < NGROUP:
            idx_start(g + 1, 1 - slot)
        g_start(slot, 0, 0)
        g_start(slot, 1, 1)

        def ibody(it, carry, slot=slot):
            for k in (0, 1):
                j = it * 2 + k
                g_wait(slot, j, k)
                s_start(slot, j, k)
                s_wait(slot, j, k)
                g_start(slot, j + 2, k)
            return carry

        lax.fori_loop(0, GROUP // 2 - 1, ibody, 0)
        for j in (GROUP - 2, GROUP - 1):
            b = j % 2
            g_wait(slot, j, b)
            s_start(slot, j, b)
            s_wait(slot, j, b)

    plsc.subcore_barrier()
    pltpu.sync_copy(acc_sh.at[pl.ds(s * RPT, RPT)],
                    out_hbm.at[c, pl.ds(s * RPT, RPT)])


@functools.cache
def _scat_call():
    return pl.kernel(
        _scat_body,
        out_type=jax.ShapeDtypeStruct((NC, N_PAD, D), jnp.float32),
        mesh=plsc.VectorSubcoreMesh(**_MESH),
        scratch_types=[
            pltpu.VMEM((GROUP, CHUNK), jnp.int32),
            pltpu.VMEM((GROUP, CHUNK), jnp.int32),
            pltpu.VMEM((GROUP, CHUNK), jnp.int32),
            pltpu.VMEM((GROUP, CHUNK), jnp.int32),
            pltpu.VMEM((CHUNK, D), jnp.float32),
            pltpu.VMEM((CHUNK, D), jnp.float32),
            pltpu.VMEM_SHARED((N_PAD, D), jnp.float32),
            pltpu.SemaphoreType.DMA,
            pltpu.SemaphoreType.DMA,
            pltpu.SemaphoreType.DMA,
            pltpu.SemaphoreType.DMA,
            pltpu.SemaphoreType.DMA,
            pltpu.SemaphoreType.DMA,
        ],
    )


_R = 1024
_GRID = N_PAD // _R


def _dinv_of(degb):
    return jnp.where(degb > 0.0, lax.rsqrt(jnp.maximum(degb, 1.0)), 0.0)


def _proj_body(x_ref, degb_ref, wu_ref, bu_ref, wf_ref, bf_ref, w1_ref,
               out_ref):
    i = pl.program_id(0)
    x = x_ref[...]
    hu = jnp.maximum(jnp.dot(x, wu_ref[...],
                             preferred_element_type=jnp.float32)
                     + bu_ref[...], 0.0)
    hf = jnp.maximum(jnp.dot(x, wf_ref[...],
                             preferred_element_type=jnp.float32)
                     + bf_ref[...], 0.0)
    row = i * _R + lax.broadcasted_iota(jnp.int32, (_R, 1), 0)
    h = jnp.where(row < N_USERS, hu, hf)
    m = jnp.dot(h, w1_ref[...], preferred_element_type=jnp.float32)
    out_ref[...] = _dinv_of(degb_ref[...]) * m


def _proj_call(x, degb, wu, bu, wf, bf, w1):
    return pl.pallas_call(
        _proj_body,
        grid=(_GRID,),
        in_specs=[
            pl.BlockSpec((_R, D), lambda i: (i, 0)),
            pl.BlockSpec((_R, D), lambda i: (i, 0)),
            pl.BlockSpec((D, D), lambda i: (0, 0)),
            pl.BlockSpec((1, D), lambda i: (0, 0)),
            pl.BlockSpec((D, D), lambda i: (0, 0)),
            pl.BlockSpec((1, D), lambda i: (0, 0)),
            pl.BlockSpec((D, D), lambda i: (0, 0)),
        ],
        out_specs=pl.BlockSpec((_R, D), lambda i: (i, 0)),
        out_shape=jax.ShapeDtypeStruct((N_PAD, D), jnp.float32),
    )(x, degb, wu, bu, wf, bf, w1)


def _mid_body(acc_ref, degb_ref, b_ref, w_ref, out_ref):
    dinv = _dinv_of(degb_ref[...])
    agg = acc_ref[0] + acc_ref[1]
    h = jnp.maximum(dinv * agg + b_ref[...], 0.0)
    m = jnp.dot(h, w_ref[...], preferred_element_type=jnp.float32)
    out_ref[...] = dinv * m


def _mid_call(acc, degb, b, w):
    return pl.pallas_call(
        _mid_body,
        grid=(_GRID,),
        in_specs=[
            pl.BlockSpec((NC, _R, D), lambda i: (0, i, 0)),
            pl.BlockSpec((_R, D), lambda i: (i, 0)),
            pl.BlockSpec((1, D), lambda i: (0, 0)),
            pl.BlockSpec((D, D), lambda i: (0, 0)),
        ],
        out_specs=pl.BlockSpec((_R, D), lambda i: (i, 0)),
        out_shape=jax.ShapeDtypeStruct((N_PAD, D), jnp.float32),
    )(acc, degb, b, w)


def _fin_body(acc_ref, degb_ref, b_ref, out_ref):
    dinv = _dinv_of(degb_ref[...])
    agg = acc_ref[0] + acc_ref[1]
    out_ref[...] = jnp.maximum(dinv * agg + b_ref[...], 0.0)


_FR = 1000


def _fin_call(acc, degb, b, n_rows, row0):
    blk0 = row0 // _FR
    return pl.pallas_call(
        _fin_body,
        grid=(n_rows // _FR,),
        in_specs=[
            pl.BlockSpec((NC, _FR, D), lambda i: (0, blk0 + i, 0)),
            pl.BlockSpec((_FR, D), lambda i: (blk0 + i, 0)),
            pl.BlockSpec((1, D), lambda i: (0, 0)),
        ],
        out_specs=pl.BlockSpec((_FR, D), lambda i: (i, 0)),
        out_shape=jax.ShapeDtypeStruct((n_rows, D), jnp.float32),
    )(acc, degb, b)


def kernel(user_x, food_x, edge_index, W_user, b_user, W_food, b_food,
           W1, b1, W2, b2):
    src = edge_index[0]
    dst = edge_index[1]
    pad_idx = N_NODES + (jnp.arange(E_PAD - E, dtype=jnp.int32)
                         % (N_PAD - N_NODES))
    src_r = jnp.concatenate([src, pad_idx]).reshape(NW, NCHUNK, CHUNK)
    dst_r = jnp.concatenate([dst, pad_idx]).reshape(NW, NCHUNK, CHUNK)
    x_pad = jnp.concatenate(
        [user_x, food_x, jnp.zeros((N_PAD - N_NODES, D), jnp.float32)], axis=0)
    zeros1 = jnp.zeros((N_PAD,), jnp.float32)
    zeros2 = jnp.zeros((N_PAD, D), jnp.float32)

    degp = _deg_call()(dst_r, zeros1)
    degb = jnp.broadcast_to((degp[0] + degp[1])[:, None], (N_PAD, D))

    m1 = _proj_call(x_pad, degb, W_user, b_user.reshape(1, D),
                    W_food, b_food.reshape(1, D), W1)
    acc1 = _scat_call()(m1, src_r, dst_r, zeros2)
    m2 = _mid_call(acc1, degb, b1.reshape(1, D), W2)
    acc2 = _scat_call()(m2, src_r, dst_r, zeros2)
    users = _fin_call(acc2, degb, b2.reshape(1, D), N_USERS, 0)
    items = _fin_call(acc2, degb, b2.reshape(1, D), N_ITEMS, N_USERS)
    return (users, users, items, items)


# speedup vs baseline: 24.4240x; 1.0417x over previous
import functools

import jax
import jax.numpy as jnp
from jax import lax
from jax.experimental import pallas as pl
from jax.experimental.pallas import tpu as pltpu
from jax.experimental.pallas import tpu_sc as plsc

N_USERS = 2000
N_ITEMS = 8000
N_NODES = N_USERS + N_ITEMS
D = 128
E = 320000

NC = 2
NS = 16
NW = NC * NS
CHUNK = 128
NCHUNK = 80
E_PAD = NW * NCHUNK * CHUNK
N_PAD = 10240
RPT = N_PAD // NS

_MESH = dict(core_axis_name="c", subcore_axis_name="s")


def _deg_body(dstr_hbm, zeros1_hbm, out_hbm, dst_v, ones_v, acc_sh):
    c = lax.axis_index("c")
    s = lax.axis_index("s")
    w = c * NS + s
    pltpu.sync_copy(zeros1_hbm.at[pl.ds(s * RPT, RPT)],
                    acc_sh.at[pl.ds(s * RPT, RPT)])
    pltpu.sync_copy(dstr_hbm.at[w], dst_v)
    for j in range(CHUNK // 16):
        ones_v[pl.ds(j * 16, 16)] = jnp.full((16,), 1.0, dtype=jnp.float32)
    plsc.subcore_barrier()

    def body(i, carry):
        pltpu.sync_copy(ones_v, acc_sh.at[dst_v.at[i]], add=True)
        return carry

    lax.fori_loop(0, NCHUNK, body, 0)
    plsc.subcore_barrier()
    pltpu.sync_copy(acc_sh.at[pl.ds(s * RPT, RPT)],
                    out_hbm.at[c, pl.ds(s * RPT, RPT)])


@functools.cache
def _deg_call():
    return pl.kernel(
        _deg_body,
        out_type=jax.ShapeDtypeStruct((NC, N_PAD), jnp.float32),
        mesh=plsc.VectorSubcoreMesh(**_MESH),
        scratch_types=[
            pltpu.VMEM((NCHUNK, CHUNK), jnp.int32),
            pltpu.VMEM((CHUNK,), jnp.float32),
            pltpu.VMEM_SHARED((N_PAD,), jnp.float32),
        ],
    )


GROUP = 16
NGROUP = NCHUNK // GROUP


def _scat_body(table_hbm, srcr_hbm, dstr_hbm, zeros2_hbm, out_hbm,
               srcs0, srcs1, dsts0, dsts1, rows0, rows1, acc_sh,
               i0sem, i1sem, g0, g1, s0, s1):
    c = lax.axis_index("c")
    s = lax.axis_index("s")
    w = c * NS + s

    src_s = (srcs0, srcs1)
    dst_s = (dsts0, dsts1)
    isem = (i0sem, i1sem)
    bufs = (rows0, rows1)
    gsem = (g0, g1)
    ssem = (s0, s1)

    def idx_start(g, slot):
        pltpu.async_copy(srcr_hbm.at[w, pl.ds(g * GROUP, GROUP)],
                         src_s[slot], isem[slot])
        pltpu.async_copy(dstr_hbm.at[w, pl.ds(g * GROUP, GROUP)],
                         dst_s[slot], isem[slot])

    def idx_wait(g, slot):
        pltpu.make_async_copy(srcr_hbm.at[w, pl.ds(g * GROUP, GROUP)],
                              src_s[slot], isem[slot]).wait()
        pltpu.make_async_copy(dstr_hbm.at[w, pl.ds(g * GROUP, GROUP)],
                              dst_s[slot], isem[slot]).wait()

    def g_start(slot, j, b):
        pltpu.async_copy(table_hbm.at[src_s[slot].at[j]], bufs[b], gsem[b])

    def g_wait(slot, j, b):
        pltpu.make_async_copy(table_hbm.at[src_s[slot].at[j]], bufs[b],
                              gsem[b]).wait()

    def s_start(slot, j, b):
        pltpu.async_copy(bufs[b], acc_sh.at[dst_s[slot].at[j]], ssem[b],
                         add=True)

    def s_wait(slot, j, b):
        pltpu.make_async_copy(bufs[b], acc_sh.at[dst_s[slot].at[j]],
                              ssem[b]).wait()

    idx_start(0, 0)
    idx_wait(0, 0)
    g_start(0, 0, 0)
    g_start(0, 1, 1)
    pltpu.sync_copy(zeros2_hbm.at[pl.ds(s * RPT, RPT)],
                    acc_sh.at[pl.ds(s * RPT, RPT)])
    plsc.subcore_barrier()

    for g in range(NGROUP):
        slot = g % 2
        if g + 1 < NGROUP:
            idx_start(g + 1, 1 - slot)

        def ibody(it, carry, slot=slot):
            for k in (0, 1):
                j = it * 2 + k
                g_wait(slot, j, k)
                s_start(slot, j, k)
                s_wait(slot, j, k)
                g_start(slot, j + 2, k)
            return carry

        lax.fori_loop(0, GROUP // 2 - 1, ibody, 0)
        for j in (GROUP - 2, GROUP - 1):
            b = j % 2
            g_wait(slot, j, b)
            s_start(slot, j, b)
            s_wait(slot, j, b)
            if g + 1 < NGROUP:
                if j == GROUP - 2:
                    idx_wait(g + 1, 1 - slot)
                g_start(1 - slot, j - (GROUP - 2), b)

    plsc.subcore_barrier()
    pltpu.sync_copy(acc_sh.at[pl.ds(s * RPT, RPT)],
                    out_hbm.at[c, pl.ds(s * RPT, RPT)])


@functools.cache
def _scat_call():
    return pl.kernel(
        _scat_body,
        out_type=jax.ShapeDtypeStruct((NC, N_PAD, D), jnp.float32),
        mesh=plsc.VectorSubcoreMesh(**_MESH),
        scratch_types=[
            pltpu.VMEM((GROUP, CHUNK), jnp.int32),
            pltpu.VMEM((GROUP, CHUNK), jnp.int32),
            pltpu.VMEM((GROUP, CHUNK), jnp.int32),
            pltpu.VMEM((GROUP, CHUNK), jnp.int32),
            pltpu.VMEM((CHUNK, D), jnp.float32),
            pltpu.VMEM((CHUNK, D), jnp.float32),
            pltpu.VMEM_SHARED((N_PAD, D), jnp.float32),
            pltpu.SemaphoreType.DMA,
            pltpu.SemaphoreType.DMA,
            pltpu.SemaphoreType.DMA,
            pltpu.SemaphoreType.DMA,
            pltpu.SemaphoreType.DMA,
            pltpu.SemaphoreType.DMA,
        ],
    )


_R = 1024
_GRID = N_PAD // _R


def _dinv_of(degb):
    return jnp.where(degb > 0.0, lax.rsqrt(jnp.maximum(degb, 1.0)), 0.0)


def _proj_body(x_ref, wu_ref, bu_ref, wf_ref, bf_ref, w1_ref, out_ref):
    i = pl.program_id(0)
    x = x_ref[...]
    hu = jnp.maximum(jnp.dot(x, wu_ref[...],
                             preferred_element_type=jnp.float32)
                     + bu_ref[...], 0.0)
    hf = jnp.maximum(jnp.dot(x, wf_ref[...],
                             preferred_element_type=jnp.float32)
                     + bf_ref[...], 0.0)
    row = i * _R + lax.broadcasted_iota(jnp.int32, (_R, 1), 0)
    h = jnp.where(row < N_USERS, hu, hf)
    out_ref[...] = jnp.dot(h, w1_ref[...], preferred_element_type=jnp.float32)


def _proj_call(x, wu, bu, wf, bf, w1):
    return pl.pallas_call(
        _proj_body,
        grid=(_GRID,),
        in_specs=[
            pl.BlockSpec((_R, D), lambda i: (i, 0)),
            pl.BlockSpec((D, D), lambda i: (0, 0)),
            pl.BlockSpec((1, D), lambda i: (0, 0)),
            pl.BlockSpec((D, D), lambda i: (0, 0)),
            pl.BlockSpec((1, D), lambda i: (0, 0)),
            pl.BlockSpec((D, D), lambda i: (0, 0)),
        ],
        out_specs=pl.BlockSpec((_R, D), lambda i: (i, 0)),
        out_shape=jax.ShapeDtypeStruct((N_PAD, D), jnp.float32),
    )(x, wu, bu, wf, bf, w1)


def _scale_body(m_ref, degb_ref, out_ref):
    out_ref[...] = _dinv_of(degb_ref[...]) * m_ref[...]


def _scale_call(m, degb):
    return pl.pallas_call(
        _scale_body,
        grid=(_GRID,),
        in_specs=[
            pl.BlockSpec((_R, D), lambda i: (i, 0)),
            pl.BlockSpec((_R, D), lambda i: (i, 0)),
        ],
        out_specs=pl.BlockSpec((_R, D), lambda i: (i, 0)),
        out_shape=jax.ShapeDtypeStruct((N_PAD, D), jnp.float32),
    )(m, degb)


def _mid_body(acc_ref, degb_ref, b_ref, w_ref, out_ref):
    dinv = _dinv_of(degb_ref[...])
    agg = acc_ref[0] + acc_ref[1]
    h = jnp.maximum(dinv * agg + b_ref[...], 0.0)
    m = jnp.dot(h, w_ref[...], preferred_element_type=jnp.float32)
    out_ref[...] = dinv * m


def _mid_call(acc, degb, b, w):
    return pl.pallas_call(
        _mid_body,
        grid=(_GRID,),
        in_specs=[
            pl.BlockSpec((NC, _R, D), lambda i: (0, i, 0)),
            pl.BlockSpec((_R, D), lambda i: (i, 0)),
            pl.BlockSpec((1, D), lambda i: (0, 0)),
            pl.BlockSpec((D, D), lambda i: (0, 0)),
        ],
        out_specs=pl.BlockSpec((_R, D), lambda i: (i, 0)),
        out_shape=jax.ShapeDtypeStruct((N_PAD, D), jnp.float32),
    )(acc, degb, b, w)


def _fin_body(acc_ref, degb_ref, b_ref, out_ref, out2_ref):
    dinv = _dinv_of(degb_ref[...])
    agg = acc_ref[0] + acc_ref[1]
    h = jnp.maximum(dinv * agg + b_ref[...], 0.0)
    out_ref[...] = h
    out2_ref[...] = h


_FR = 1000


def _fin_call(acc, degb, b, n_rows, row0):
    blk0 = row0 // _FR
    return pl.pallas_call(
        _fin_body,
        grid=(n_rows // _FR,),
        in_specs=[
            pl.BlockSpec((NC, _FR, D), lambda i: (0, blk0 + i, 0)),
            pl.BlockSpec((_FR, D), lambda i: (blk0 + i, 0)),
            pl.BlockSpec((1, D), lambda i: (0, 0)),
        ],
        out_specs=[
            pl.BlockSpec((_FR, D), lambda i: (i, 0)),
            pl.BlockSpec((_FR, D), lambda i: (i, 0)),
        ],
        out_shape=[
            jax.ShapeDtypeStruct((n_rows, D), jnp.float32),
            jax.ShapeDtypeStruct((n_rows, D), jnp.float32),
        ],
    )(acc, degb, b)


def kernel(user_x, food_x, edge_index, W_user, b_user, W_food, b_food,
           W1, b1, W2, b2):
    src = edge_index[0]
    dst = edge_index[1]
    pad_idx = N_NODES + (jnp.arange(E_PAD - E, dtype=jnp.int32)
                         % (N_PAD - N_NODES))
    dst_r = jnp.concatenate([dst, pad_idx]).reshape(NW, NCHUNK, CHUNK)
    src_r = jnp.concatenate([src, pad_idx]).reshape(NW, NCHUNK, CHUNK)
    x_pad = jnp.concatenate(
        [user_x, food_x, jnp.zeros((N_PAD - N_NODES, D), jnp.float32)], axis=0)
    zeros1 = jnp.zeros((N_PAD,), jnp.float32)
    zeros2 = jnp.zeros((N_PAD, D), jnp.float32)

    degp = _deg_call()(dst_r, zeros1)
    degb = jnp.broadcast_to((degp[0] + degp[1])[:, None], (N_PAD, D))

    m1r = _proj_call(x_pad, W_user, b_user.reshape(1, D),
                     W_food, b_food.reshape(1, D), W1)
    m1 = _scale_call(m1r, degb)
    acc1 = _scat_call()(m1, src_r, dst_r, zeros2)
    m2 = _mid_call(acc1, degb, b1.reshape(1, D), W2)
    acc2 = _scat_call()(m2, src_r, dst_r, zeros2)
    users, users2 = _fin_call(acc2, degb, b2.reshape(1, D), N_USERS, 0)
    items, items2 = _fin_call(acc2, degb, b2.reshape(1, D), N_ITEMS, N_USERS)
    return (users, users2, items, items2)


# speedup vs baseline: 25.1424x; 1.0294x over previous
import functools

import jax
import jax.numpy as jnp
import numpy as np
from jax import lax
from jax.experimental import pallas as pl
from jax.experimental.pallas import tpu as pltpu
from jax.experimental.pallas import tpu_sc as plsc

N_USERS = 2000
N_ITEMS = 8000
N_NODES = N_USERS + N_ITEMS
D = 128
E = 320000

NC = 2
NS = 16
NW = NC * NS
CHUNK = 128
NCHUNK = 80
E_PAD = NW * NCHUNK * CHUNK
N_PAD = 10240
RPT = N_PAD // NS

_MESH = dict(core_axis_name="c", subcore_axis_name="s")


def _deg_body(dstr_hbm, zeros1_hbm, out_hbm, dst_v, ones_v, acc_sh, dsem):
    c = lax.axis_index("c")
    s = lax.axis_index("s")
    w = c * NS + s
    pltpu.sync_copy(zeros1_hbm.at[pl.ds(s * RPT, RPT)],
                    acc_sh.at[pl.ds(s * RPT, RPT)])
    pltpu.sync_copy(dstr_hbm.at[w], dst_v)
    for j in range(CHUNK // 16):
        ones_v[pl.ds(j * 16, 16)] = jnp.full((16,), 1.0, dtype=jnp.float32)
    plsc.subcore_barrier()

    WAVE = 16

    def body(iw, carry):
        for k in range(WAVE):
            pltpu.async_copy(ones_v, acc_sh.at[dst_v.at[iw * WAVE + k]],
                             dsem, add=True)
        for k in range(WAVE):
            pltpu.make_async_copy(ones_v, acc_sh.at[dst_v.at[iw * WAVE + k]],
                                  dsem).wait()
        return carry

    lax.fori_loop(0, NCHUNK // WAVE, body, 0)
    plsc.subcore_barrier()
    pltpu.sync_copy(acc_sh.at[pl.ds(s * RPT, RPT)],
                    out_hbm.at[c, pl.ds(s * RPT, RPT)])


@functools.cache
def _deg_call():
    return pl.kernel(
        _deg_body,
        out_type=jax.ShapeDtypeStruct((NC, N_PAD), jnp.float32),
        mesh=plsc.VectorSubcoreMesh(**_MESH),
        scratch_types=[
            pltpu.VMEM((NCHUNK, CHUNK), jnp.int32),
            pltpu.VMEM((CHUNK,), jnp.float32),
            pltpu.VMEM_SHARED((N_PAD,), jnp.float32),
            pltpu.SemaphoreType.DMA,
        ],
    )


GROUP = 16
NGROUP = NCHUNK // GROUP


def _scat_body(table_hbm, srcr_hbm, dstr_hbm, zeros2_hbm, out_hbm,
               srcs0, srcs1, dsts0, dsts1, rows0, rows1, acc_sh,
               i0sem, i1sem, g0, g1, s0, s1):
    c = lax.axis_index("c")
    s = lax.axis_index("s")
    w = c * NS + s

    src_s = (srcs0, srcs1)
    dst_s = (dsts0, dsts1)
    isem = (i0sem, i1sem)
    bufs = (rows0, rows1)
    gsem = (g0, g1)
    ssem = (s0, s1)

    def idx_start(g, slot):
        pltpu.async_copy(srcr_hbm.at[w, pl.ds(g * GROUP, GROUP)],
                         src_s[slot], isem[slot])
        pltpu.async_copy(dstr_hbm.at[w, pl.ds(g * GROUP, GROUP)],
                         dst_s[slot], isem[slot])

    def idx_wait(g, slot):
        pltpu.make_async_copy(srcr_hbm.at[w, pl.ds(g * GROUP, GROUP)],
                              src_s[slot], isem[slot]).wait()
        pltpu.make_async_copy(dstr_hbm.at[w, pl.ds(g * GROUP, GROUP)],
                              dst_s[slot], isem[slot]).wait()

    def g_start(slot, j, b):
        pltpu.async_copy(table_hbm.at[src_s[slot].at[j]], bufs[b], gsem[b])

    def g_wait(slot, j, b):
        pltpu.make_async_copy(table_hbm.at[src_s[slot].at[j]], bufs[b],
                              gsem[b]).wait()

    def s_start(slot, j, b):
        pltpu.async_copy(bufs[b], acc_sh.at[dst_s[slot].at[j]], ssem[b],
                         add=True)

    def s_wait(slot, j, b):
        pltpu.make_async_copy(bufs[b], acc_sh.at[dst_s[slot].at[j]],
                              ssem[b]).wait()

    idx_start(0, 0)
    idx_wait(0, 0)
    g_start(0, 0, 0)
    g_start(0, 1, 1)
    pltpu.sync_copy(zeros2_hbm.at[pl.ds(s * RPT, RPT)],
                    acc_sh.at[pl.ds(s * RPT, RPT)])
    plsc.subcore_barrier()

    for g in range(NGROUP):
        slot = g % 2
        if g + 1 < NGROUP:
            idx_start(g + 1, 1 - slot)

        def ibody(it, carry, slot=slot):
            for k in (0, 1):
                j = it * 2 + k
                g_wait(slot, j, k)
                s_start(slot, j, k)
                s_wait(slot, j, k)
                g_start(slot, j + 2, k)
            return carry

        lax.fori_loop(0, GROUP // 2 - 1, ibody, 0)
        for j in (GROUP - 2, GROUP - 1):
            b = j % 2
            g_wait(slot, j, b)
            s_start(slot, j, b)
            s_wait(slot, j, b)
            if g + 1 < NGROUP:
                if j == GROUP - 2:
                    idx_wait(g + 1, 1 - slot)
                g_start(1 - slot, j - (GROUP - 2), b)

    plsc.subcore_barrier()
    pltpu.sync_copy(acc_sh.at[pl.ds(s * RPT, RPT)],
                    out_hbm.at[c, pl.ds(s * RPT, RPT)])


@functools.cache
def _scat_call():
    return pl.kernel(
        _scat_body,
        out_type=jax.ShapeDtypeStruct((NC, N_PAD, D), jnp.float32),
        mesh=plsc.VectorSubcoreMesh(**_MESH),
        scratch_types=[
            pltpu.VMEM((GROUP, CHUNK), jnp.int32),
            pltpu.VMEM((GROUP, CHUNK), jnp.int32),
            pltpu.VMEM((GROUP, CHUNK), jnp.int32),
            pltpu.VMEM((GROUP, CHUNK), jnp.int32),
            pltpu.VMEM((CHUNK, D), jnp.float32),
            pltpu.VMEM((CHUNK, D), jnp.float32),
            pltpu.VMEM_SHARED((N_PAD, D), jnp.float32),
            pltpu.SemaphoreType.DMA,
            pltpu.SemaphoreType.DMA,
            pltpu.SemaphoreType.DMA,
            pltpu.SemaphoreType.DMA,
            pltpu.SemaphoreType.DMA,
            pltpu.SemaphoreType.DMA,
        ],
    )


_R = 1024
_GRID = N_PAD // _R


def _dinv_of(degb):
    return jnp.where(degb > 0.0, lax.rsqrt(jnp.maximum(degb, 1.0)), 0.0)


def _proj_body(x_ref, degb_ref, wu_ref, bu_ref, wf_ref, bf_ref, w1_ref,
               out_ref):
    i = pl.program_id(0)
    x = x_ref[...]
    hu = jnp.maximum(jnp.dot(x, wu_ref[...],
                             preferred_element_type=jnp.float32)
                     + bu_ref[...], 0.0)
    hf = jnp.maximum(jnp.dot(x, wf_ref[...],
                             preferred_element_type=jnp.float32)
                     + bf_ref[...], 0.0)
    row = i * _R + lax.broadcasted_iota(jnp.int32, (_R, 1), 0)
    h = jnp.where(row < N_USERS, hu, hf)
    m = jnp.dot(h, w1_ref[...], preferred_element_type=jnp.float32)
    out_ref[...] = _dinv_of(degb_ref[...]) * m


def _proj_call(x, degb, wu, bu, wf, bf, w1):
    return pl.pallas_call(
        _proj_body,
        grid=(_GRID,),
        in_specs=[
            pl.BlockSpec((_R, D), lambda i: (i, 0)),
            pl.BlockSpec((_R, D), lambda i: (i, 0)),
            pl.BlockSpec((D, D), lambda i: (0, 0)),
            pl.BlockSpec((1, D), lambda i: (0, 0)),
            pl.BlockSpec((D, D), lambda i: (0, 0)),
            pl.BlockSpec((1, D), lambda i: (0, 0)),
            pl.BlockSpec((D, D), lambda i: (0, 0)),
        ],
        out_specs=pl.BlockSpec((_R, D), lambda i: (i, 0)),
        out_shape=jax.ShapeDtypeStruct((N_PAD, D), jnp.float32),
    )(x, degb, wu, bu, wf, bf, w1)


def _mid_body(acc_ref, degb_ref, b_ref, w_ref, out_ref):
    dinv = _dinv_of(degb_ref[...])
    agg = acc_ref[0] + acc_ref[1]
    h = jnp.maximum(dinv * agg + b_ref[...], 0.0)
    m = jnp.dot(h, w_ref[...], preferred_element_type=jnp.float32)
    out_ref[...] = dinv * m


def _mid_call(acc, degb, b, w):
    return pl.pallas_call(
        _mid_body,
        grid=(_GRID,),
        in_specs=[
            pl.BlockSpec((NC, _R, D), lambda i: (0, i, 0)),
            pl.BlockSpec((_R, D), lambda i: (i, 0)),
            pl.BlockSpec((1, D), lambda i: (0, 0)),
            pl.BlockSpec((D, D), lambda i: (0, 0)),
        ],
        out_specs=pl.BlockSpec((_R, D), lambda i: (i, 0)),
        out_shape=jax.ShapeDtypeStruct((N_PAD, D), jnp.float32),
    )(acc, degb, b, w)


def _fin_body(acc_ref, degb_ref, b_ref, out_ref, out2_ref):
    dinv = _dinv_of(degb_ref[...])
    agg = acc_ref[0] + acc_ref[1]
    h = jnp.maximum(dinv * agg + b_ref[...], 0.0)
    out_ref[...] = h
    out2_ref[...] = h


_FR = 2000


def _fin_call(acc, degb, b, n_rows, row0):
    blk0 = row0 // _FR
    return pl.pallas_call(
        _fin_body,
        grid=(n_rows // _FR,),
        in_specs=[
            pl.BlockSpec((NC, _FR, D), lambda i: (0, blk0 + i, 0)),
            pl.BlockSpec((_FR, D), lambda i: (blk0 + i, 0)),
            pl.BlockSpec((1, D), lambda i: (0, 0)),
        ],
        out_specs=[
            pl.BlockSpec((_FR, D), lambda i: (i, 0)),
            pl.BlockSpec((_FR, D), lambda i: (i, 0)),
        ],
        out_shape=[
            jax.ShapeDtypeStruct((n_rows, D), jnp.float32),
            jax.ShapeDtypeStruct((n_rows, D), jnp.float32),
        ],
    )(acc, degb, b)


def kernel(user_x, food_x, edge_index, W_user, b_user, W_food, b_food,
           W1, b1, W2, b2):
    src = edge_index[0]
    dst = edge_index[1]
    pad_idx = jnp.asarray(
        np.asarray(N_NODES + np.arange(E_PAD - E) % (N_PAD - N_NODES),
                   dtype=np.int32))
    dst_r = jnp.concatenate([dst, pad_idx]).reshape(NW, NCHUNK, CHUNK)
    src_r = jnp.concatenate([src, pad_idx]).reshape(NW, NCHUNK, CHUNK)
    x_pad = jnp.concatenate(
        [user_x, food_x, jnp.zeros((N_PAD - N_NODES, D), jnp.float32)], axis=0)
    zeros1 = jnp.zeros((N_PAD,), jnp.float32)
    zeros2 = jnp.zeros((N_PAD, D), jnp.float32)

    degp = _deg_call()(dst_r, zeros1)
    degb = jnp.broadcast_to((degp[0] + degp[1])[:, None], (N_PAD, D))

    m1 = _proj_call(x_pad, degb, W_user, b_user.reshape(1, D),
                    W_food, b_food.reshape(1, D), W1)
    acc1 = _scat_call()(m1, src_r, dst_r, zeros2)
    m2 = _mid_call(acc1, degb, b1.reshape(1, D), W2)
    acc2 = _scat_call()(m2, src_r, dst_r, zeros2)
    users, users2 = _fin_call(acc2, degb, b2.reshape(1, D), N_USERS, 0)
    items, items2 = _fin_call(acc2, degb, b2.reshape(1, D), N_ITEMS, N_USERS)
    return (users, users2, items, items2)


# speedup vs baseline: 25.6380x; 1.0197x over previous
import functools

import jax
import jax.numpy as jnp
import numpy as np
from jax import lax
from jax.experimental import pallas as pl
from jax.experimental.pallas import tpu as pltpu
from jax.experimental.pallas import tpu_sc as plsc

N_USERS = 2000
N_ITEMS = 8000
N_NODES = N_USERS + N_ITEMS
D = 128
E = 320000

NC = 2
NS = 16
NW = NC * NS
CHUNK = 128
NCHUNK = 80
E_PAD = NW * NCHUNK * CHUNK
N_PAD = 10240
RPT = N_PAD // NS

_MESH = dict(core_axis_name="c", subcore_axis_name="s")


def _deg_body(dstr_hbm, zeros1_hbm, out_hbm, dst_v, ones_v, acc_sh, dsem):
    c = lax.axis_index("c")
    s = lax.axis_index("s")
    w = c * NS + s
    pltpu.sync_copy(zeros1_hbm.at[pl.ds(s * RPT, RPT)],
                    acc_sh.at[pl.ds(s * RPT, RPT)])
    pltpu.sync_copy(dstr_hbm.at[w], dst_v)
    for j in range(CHUNK // 16):
        ones_v[pl.ds(j * 16, 16)] = jnp.full((16,), 1.0, dtype=jnp.float32)
    plsc.subcore_barrier()

    WAVE = 16

    def body(iw, carry):
        for k in range(WAVE):
            pltpu.async_copy(ones_v, acc_sh.at[dst_v.at[iw * WAVE + k]],
                             dsem, add=True)
        for k in range(WAVE):
            pltpu.make_async_copy(ones_v, acc_sh.at[dst_v.at[iw * WAVE + k]],
                                  dsem).wait()
        return carry

    lax.fori_loop(0, NCHUNK // WAVE, body, 0)
    plsc.subcore_barrier()
    pltpu.sync_copy(acc_sh.at[pl.ds(s * RPT, RPT)],
                    out_hbm.at[c, pl.ds(s * RPT, RPT)])


@functools.cache
def _deg_call():
    return pl.kernel(
        _deg_body,
        out_type=jax.ShapeDtypeStruct((NC, N_PAD), jnp.float32),
        mesh=plsc.VectorSubcoreMesh(**_MESH),
        scratch_types=[
            pltpu.VMEM((NCHUNK, CHUNK), jnp.int32),
            pltpu.VMEM((CHUNK,), jnp.float32),
            pltpu.VMEM_SHARED((N_PAD,), jnp.float32),
            pltpu.SemaphoreType.DMA,
        ],
    )


GROUP = 16
NGROUP = NCHUNK // GROUP


def _scat_body(table_hbm, srcr_hbm, dstr_hbm, zeros2_hbm, out_hbm,
               srcs0, srcs1, dsts0, dsts1, rows0, rows1, acc_sh,
               i0sem, i1sem, g0, g1, s0, s1, zsem):
    c = lax.axis_index("c")
    s = lax.axis_index("s")
    w = c * NS + s

    src_s = (srcs0, srcs1)
    dst_s = (dsts0, dsts1)
    isem = (i0sem, i1sem)
    bufs = (rows0, rows1)
    gsem = (g0, g1)
    ssem = (s0, s1)

    def idx_start(g, slot):
        pltpu.async_copy(srcr_hbm.at[w, pl.ds(g * GROUP, GROUP)],
                         src_s[slot], isem[slot])
        pltpu.async_copy(dstr_hbm.at[w, pl.ds(g * GROUP, GROUP)],
                         dst_s[slot], isem[slot])

    def idx_wait(g, slot):
        pltpu.make_async_copy(srcr_hbm.at[w, pl.ds(g * GROUP, GROUP)],
                              src_s[slot], isem[slot]).wait()
        pltpu.make_async_copy(dstr_hbm.at[w, pl.ds(g * GROUP, GROUP)],
                              dst_s[slot], isem[slot]).wait()

    def g_start(slot, j, b):
        pltpu.async_copy(table_hbm.at[src_s[slot].at[j]], bufs[b], gsem[b])

    def g_wait(slot, j, b):
        pltpu.make_async_copy(table_hbm.at[src_s[slot].at[j]], bufs[b],
                              gsem[b]).wait()

    def s_start(slot, j, b):
        pltpu.async_copy(bufs[b], acc_sh.at[dst_s[slot].at[j]], ssem[b],
                         add=True)

    def s_wait(slot, j, b):
        pltpu.make_async_copy(bufs[b], acc_sh.at[dst_s[slot].at[j]],
                              ssem[b]).wait()

    pltpu.async_copy(zeros2_hbm.at[pl.ds(s * RPT, RPT)],
                     acc_sh.at[pl.ds(s * RPT, RPT)], zsem)
    idx_start(0, 0)
    idx_wait(0, 0)
    g_start(0, 0, 0)
    g_start(0, 1, 1)
    pltpu.make_async_copy(zeros2_hbm.at[pl.ds(s * RPT, RPT)],
                          acc_sh.at[pl.ds(s * RPT, RPT)], zsem).wait()
    plsc.subcore_barrier()

    for g in range(NGROUP):
        slot = g % 2
        if g + 1 < NGROUP:
            idx_start(g + 1, 1 - slot)

        def ibody(it, carry, slot=slot):
            for k in (0, 1):
                j = it * 2 + k
                g_wait(slot, j, k)
                s_start(slot, j, k)
                s_wait(slot, j, k)
                g_start(slot, j + 2, k)
            return carry

        lax.fori_loop(0, GROUP // 2 - 1, ibody, 0)
        for j in (GROUP - 2, GROUP - 1):
            b = j % 2
            g_wait(slot, j, b)
            s_start(slot, j, b)
            s_wait(slot, j, b)
            if g + 1 < NGROUP:
                if j == GROUP - 2:
                    idx_wait(g + 1, 1 - slot)
                g_start(1 - slot, j - (GROUP - 2), b)

    plsc.subcore_barrier()
    pltpu.sync_copy(acc_sh.at[pl.ds(s * RPT, RPT)],
                    out_hbm.at[c, pl.ds(s * RPT, RPT)])


@functools.cache
def _scat_call():
    return pl.kernel(
        _scat_body,
        out_type=jax.ShapeDtypeStruct((NC, N_PAD, D), jnp.float32),
        mesh=plsc.VectorSubcoreMesh(**_MESH),
        scratch_types=[
            pltpu.VMEM((GROUP, CHUNK), jnp.int32),
            pltpu.VMEM((GROUP, CHUNK), jnp.int32),
            pltpu.VMEM((GROUP, CHUNK), jnp.int32),
            pltpu.VMEM((GROUP, CHUNK), jnp.int32),
            pltpu.VMEM((CHUNK, D), jnp.float32),
            pltpu.VMEM((CHUNK, D), jnp.float32),
            pltpu.VMEM_SHARED((N_PAD, D), jnp.float32),
            pltpu.SemaphoreType.DMA,
            pltpu.SemaphoreType.DMA,
            pltpu.SemaphoreType.DMA,
            pltpu.SemaphoreType.DMA,
            pltpu.SemaphoreType.DMA,
            pltpu.SemaphoreType.DMA,
            pltpu.SemaphoreType.DMA,
        ],
    )


_R = 2048
_GRID = N_PAD // _R


def _dinv_of(degb):
    return jnp.where(degb > 0.0, lax.rsqrt(jnp.maximum(degb, 1.0)), 0.0)


def _proj_body(x_ref, degb_ref, wu_ref, bu_ref, wf_ref, bf_ref, w1_ref,
               out_ref):
    i = pl.program_id(0)
    x = x_ref[...]
    hu = jnp.maximum(jnp.dot(x, wu_ref[...],
                             preferred_element_type=jnp.float32)
                     + bu_ref[...], 0.0)
    hf = jnp.maximum(jnp.dot(x, wf_ref[...],
                             preferred_element_type=jnp.float32)
                     + bf_ref[...], 0.0)
    row = i * _R + lax.broadcasted_iota(jnp.int32, (_R, 1), 0)
    h = jnp.where(row < N_USERS, hu, hf)
    m = jnp.dot(h, w1_ref[...], preferred_element_type=jnp.float32)
    out_ref[...] = _dinv_of(degb_ref[...]) * m


def _proj_call(x, degb, wu, bu, wf, bf, w1):
    return pl.pallas_call(
        _proj_body,
        grid=(_GRID,),
        in_specs=[
            pl.BlockSpec((_R, D), lambda i: (i, 0)),
            pl.BlockSpec((_R, D), lambda i: (i, 0)),
            pl.BlockSpec((D, D), lambda i: (0, 0)),
            pl.BlockSpec((1, D), lambda i: (0, 0)),
            pl.BlockSpec((D, D), lambda i: (0, 0)),
            pl.BlockSpec((1, D), lambda i: (0, 0)),
            pl.BlockSpec((D, D), lambda i: (0, 0)),
        ],
        out_specs=pl.BlockSpec((_R, D), lambda i: (i, 0)),
        out_shape=jax.ShapeDtypeStruct((N_PAD, D), jnp.float32),
    )(x, degb, wu, bu, wf, bf, w1)


def _mid_body(acc_ref, degb_ref, b_ref, w_ref, out_ref):
    dinv = _dinv_of(degb_ref[...])
    agg = acc_ref[0] + acc_ref[1]
    h = jnp.maximum(dinv * agg + b_ref[...], 0.0)
    m = jnp.dot(h, w_ref[...], preferred_element_type=jnp.float32)
    out_ref[...] = dinv * m


def _mid_call(acc, degb, b, w):
    return pl.pallas_call(
        _mid_body,
        grid=(_GRID,),
        in_specs=[
            pl.BlockSpec((NC, _R, D), lambda i: (0, i, 0)),
            pl.BlockSpec((_R, D), lambda i: (i, 0)),
            pl.BlockSpec((1, D), lambda i: (0, 0)),
            pl.BlockSpec((D, D), lambda i: (0, 0)),
        ],
        out_specs=pl.BlockSpec((_R, D), lambda i: (i, 0)),
        out_shape=jax.ShapeDtypeStruct((N_PAD, D), jnp.float32),
    )(acc, degb, b, w)


def _fin_body(acc_ref, degb_ref, b_ref, out_ref, out2_ref):
    dinv = _dinv_of(degb_ref[...])
    agg = acc_ref[0] + acc_ref[1]
    h = jnp.maximum(dinv * agg + b_ref[...], 0.0)
    out_ref[...] = h
    out2_ref[...] = h


_FR = 2000


def _fin_call(acc, degb, b, n_rows, row0):
    blk0 = row0 // _FR
    return pl.pallas_call(
        _fin_body,
        grid=(n_rows // _FR,),
        in_specs=[
            pl.BlockSpec((NC, _FR, D), lambda i: (0, blk0 + i, 0)),
            pl.BlockSpec((_FR, D), lambda i: (blk0 + i, 0)),
            pl.BlockSpec((1, D), lambda i: (0, 0)),
        ],
        out_specs=[
            pl.BlockSpec((_FR, D), lambda i: (i, 0)),
            pl.BlockSpec((_FR, D), lambda i: (i, 0)),
        ],
        out_shape=[
            jax.ShapeDtypeStruct((n_rows, D), jnp.float32),
            jax.ShapeDtypeStruct((n_rows, D), jnp.float32),
        ],
    )(acc, degb, b)


def kernel(user_x, food_x, edge_index, W_user, b_user, W_food, b_food,
           W1, b1, W2, b2):
    src = edge_index[0]
    dst = edge_index[1]
    pad_idx = jnp.asarray(
        np.asarray(N_NODES + np.arange(E_PAD - E) % (N_PAD - N_NODES),
                   dtype=np.int32))
    dst_r = jnp.concatenate([dst, pad_idx]).reshape(NW, NCHUNK, CHUNK)
    src_r = jnp.concatenate([src, pad_idx]).reshape(NW, NCHUNK, CHUNK)
    x_pad = jnp.concatenate(
        [user_x, food_x, jnp.zeros((N_PAD - N_NODES, D), jnp.float32)], axis=0)
    zeros1 = jnp.asarray(np.zeros((N_PAD,), np.float32))
    zeros2 = jnp.asarray(np.zeros((N_PAD, D), np.float32))

    degp = _deg_call()(dst_r, zeros1)
    degb = jnp.broadcast_to((degp[0] + degp[1])[:, None], (N_PAD, D))

    m1 = _proj_call(x_pad, degb, W_user, b_user.reshape(1, D),
                    W_food, b_food.reshape(1, D), W1)
    acc1 = _scat_call()(m1, src_r, dst_r, zeros2)
    m2 = _mid_call(acc1, degb, b1.reshape(1, D), W2)
    acc2 = _scat_call()(m2, src_r, dst_r, zeros2)
    users, users2 = _fin_call(acc2, degb, b2.reshape(1, D), N_USERS, 0)
    items, items2 = _fin_call(acc2, degb, b2.reshape(1, D), N_ITEMS, N_USERS)
    return (users, users2, items, items2)


# speedup vs baseline: 26.0078x; 1.0144x over previous
import functools

import jax
import jax.numpy as jnp
import numpy as np
from jax import lax
from jax.experimental import pallas as pl
from jax.experimental.pallas import tpu as pltpu
from jax.experimental.pallas import tpu_sc as plsc

N_USERS = 2000
N_ITEMS = 8000
N_NODES = N_USERS + N_ITEMS
D = 128
E = 320000

NC = 2
NS = 16
NW = NC * NS
CHUNK = 128
NCHUNK = 80
E_PAD = NW * NCHUNK * CHUNK
N_PAD = 10240
RPT = N_PAD // NS

_MESH = dict(core_axis_name="c", subcore_axis_name="s")


def _deg_body(dstr_hbm, zeros1_hbm, out_hbm, dst_v, ones_v, acc_sh, dsem):
    c = lax.axis_index("c")
    s = lax.axis_index("s")
    w = c * NS + s
    pltpu.sync_copy(zeros1_hbm.at[pl.ds(s * RPT, RPT)],
                    acc_sh.at[pl.ds(s * RPT, RPT)])
    pltpu.sync_copy(dstr_hbm.at[w], dst_v)
    for j in range(CHUNK // 16):
        ones_v[pl.ds(j * 16, 16)] = jnp.full((16,), 1.0, dtype=jnp.float32)
    plsc.subcore_barrier()

    WAVE = 16

    def body(iw, carry):
        for k in range(WAVE):
            pltpu.async_copy(ones_v, acc_sh.at[dst_v.at[iw * WAVE + k]],
                             dsem, add=True)
        for k in range(WAVE):
            pltpu.make_async_copy(ones_v, acc_sh.at[dst_v.at[iw * WAVE + k]],
                                  dsem).wait()
        return carry

    lax.fori_loop(0, NCHUNK // WAVE, body, 0)
    plsc.subcore_barrier()
    pltpu.sync_copy(acc_sh.at[pl.ds(s * RPT, RPT)],
                    out_hbm.at[c, pl.ds(s * RPT, RPT)])


@functools.cache
def _deg_call():
    return pl.kernel(
        _deg_body,
        out_type=jax.ShapeDtypeStruct((NC, N_PAD), jnp.float32),
        mesh=plsc.VectorSubcoreMesh(**_MESH),
        scratch_types=[
            pltpu.VMEM((NCHUNK, CHUNK), jnp.int32),
            pltpu.VMEM((CHUNK,), jnp.float32),
            pltpu.VMEM_SHARED((N_PAD,), jnp.float32),
            pltpu.SemaphoreType.DMA,
        ],
    )


GROUP = 16
NGROUP = NCHUNK // GROUP


def _scat_body(table_hbm, srcr_hbm, dstr_hbm, zeros2_hbm, out_hbm,
               srcs0, srcs1, dsts0, dsts1, rows0, rows1, acc_sh,
               i0sem, i1sem, g0, g1, s0, s1, zsem):
    c = lax.axis_index("c")
    s = lax.axis_index("s")
    w = c * NS + s

    src_s = (srcs0, srcs1)
    dst_s = (dsts0, dsts1)
    isem = (i0sem, i1sem)
    bufs = (rows0, rows1)
    gsem = (g0, g1)
    ssem = (s0, s1)

    def idx_start(g, slot):
        pltpu.async_copy(srcr_hbm.at[w, pl.ds(g * GROUP, GROUP)],
                         src_s[slot], isem[slot])
        pltpu.async_copy(dstr_hbm.at[w, pl.ds(g * GROUP, GROUP)],
                         dst_s[slot], isem[slot])

    def idx_wait(g, slot):
        pltpu.make_async_copy(srcr_hbm.at[w, pl.ds(g * GROUP, GROUP)],
                              src_s[slot], isem[slot]).wait()
        pltpu.make_async_copy(dstr_hbm.at[w, pl.ds(g * GROUP, GROUP)],
                              dst_s[slot], isem[slot]).wait()

    def g_start(slot, j, b):
        pltpu.async_copy(table_hbm.at[src_s[slot].at[j]], bufs[b], gsem[b])

    def g_wait(slot, j, b):
        pltpu.make_async_copy(table_hbm.at[src_s[slot].at[j]], bufs[b],
                              gsem[b]).wait()

    def s_start(slot, j, b):
        pltpu.async_copy(bufs[b], acc_sh.at[dst_s[slot].at[j]], ssem[b],
                         add=True)

    def s_wait(slot, j, b):
        pltpu.make_async_copy(bufs[b], acc_sh.at[dst_s[slot].at[j]],
                              ssem[b]).wait()

    pltpu.async_copy(zeros2_hbm.at[pl.ds(s * RPT, RPT)],
                     acc_sh.at[pl.ds(s * RPT, RPT)], zsem)
    idx_start(0, 0)
    idx_wait(0, 0)
    g_start(0, 0, 0)
    g_start(0, 1, 1)
    pltpu.make_async_copy(zeros2_hbm.at[pl.ds(s * RPT, RPT)],
                          acc_sh.at[pl.ds(s * RPT, RPT)], zsem).wait()
    plsc.subcore_barrier()

    for g in range(NGROUP):
        slot = g % 2
        if g + 1 < NGROUP:
            idx_start(g + 1, 1 - slot)

        def ibody(it, carry, slot=slot):
            for k in (0, 1):
                j = it * 2 + k
                g_wait(slot, j, k)
                s_start(slot, j, k)
                s_wait(slot, j, k)
                g_start(slot, j + 2, k)
            return carry

        lax.fori_loop(0, GROUP // 2 - 1, ibody, 0)
        for j in (GROUP - 2, GROUP - 1):
            b = j % 2
            g_wait(slot, j, b)
            s_start(slot, j, b)
            s_wait(slot, j, b)
            if g + 1 < NGROUP:
                if j == GROUP - 2:
                    idx_wait(g + 1, 1 - slot)
                g_start(1 - slot, j - (GROUP - 2), b)

    plsc.subcore_barrier()
    pltpu.sync_copy(acc_sh.at[pl.ds(s * RPT, RPT)],
                    out_hbm.at[c, pl.ds(s * RPT, RPT)])


@functools.cache
def _scat_call():
    return pl.kernel(
        _scat_body,
        out_type=jax.ShapeDtypeStruct((NC, N_PAD, D), jnp.float32),
        mesh=plsc.VectorSubcoreMesh(**_MESH),
        scratch_types=[
            pltpu.VMEM((GROUP, CHUNK), jnp.int32),
            pltpu.VMEM((GROUP, CHUNK), jnp.int32),
            pltpu.VMEM((GROUP, CHUNK), jnp.int32),
            pltpu.VMEM((GROUP, CHUNK), jnp.int32),
            pltpu.VMEM((CHUNK, D), jnp.float32),
            pltpu.VMEM((CHUNK, D), jnp.float32),
            pltpu.VMEM_SHARED((N_PAD, D), jnp.float32),
            pltpu.SemaphoreType.DMA,
            pltpu.SemaphoreType.DMA,
            pltpu.SemaphoreType.DMA,
            pltpu.SemaphoreType.DMA,
            pltpu.SemaphoreType.DMA,
            pltpu.SemaphoreType.DMA,
            pltpu.SemaphoreType.DMA,
        ],
    )


_R = 2048
_GRID = N_PAD // _R


def _dinv_of(degb):
    return jnp.where(degb > 0.0, lax.rsqrt(jnp.maximum(degb, 1.0)), 0.0)


def _degb_body(degp_ref, out_ref):
    out_ref[...] = lax.dot_general(
        degp_ref[...], jnp.ones((NC, D), jnp.float32),
        (((0,), (0,)), ((), ())), preferred_element_type=jnp.float32)


def _degb_call(degp):
    return pl.pallas_call(
        _degb_body,
        grid=(_GRID,),
        in_specs=[pl.BlockSpec((NC, _R), lambda i: (0, i))],
        out_specs=pl.BlockSpec((_R, D), lambda i: (i, 0)),
        out_shape=jax.ShapeDtypeStruct((N_PAD, D), jnp.float32),
    )(degp)


def _proj_body(x_ref, degb_ref, wu_ref, bu_ref, wf_ref, bf_ref, w1_ref,
               out_ref):
    i = pl.program_id(0)
    x = x_ref[...]
    hu = jnp.maximum(jnp.dot(x, wu_ref[...],
                             preferred_element_type=jnp.float32)
                     + bu_ref[...], 0.0)
    hf = jnp.maximum(jnp.dot(x, wf_ref[...],
                             preferred_element_type=jnp.float32)
                     + bf_ref[...], 0.0)
    row = i * _R + lax.broadcasted_iota(jnp.int32, (_R, 1), 0)
    h = jnp.where(row < N_USERS, hu, hf)
    m = jnp.dot(h, w1_ref[...], preferred_element_type=jnp.float32)
    out_ref[...] = _dinv_of(degb_ref[...]) * m


def _proj_call(x, degb, wu, bu, wf, bf, w1):
    return pl.pallas_call(
        _proj_body,
        grid=(_GRID,),
        in_specs=[
            pl.BlockSpec((_R, D), lambda i: (i, 0)),
            pl.BlockSpec((_R, D), lambda i: (i, 0)),
            pl.BlockSpec((D, D), lambda i: (0, 0)),
            pl.BlockSpec((1, D), lambda i: (0, 0)),
            pl.BlockSpec((D, D), lambda i: (0, 0)),
            pl.BlockSpec((1, D), lambda i: (0, 0)),
            pl.BlockSpec((D, D), lambda i: (0, 0)),
        ],
        out_specs=pl.BlockSpec((_R, D), lambda i: (i, 0)),
        out_shape=jax.ShapeDtypeStruct((N_PAD, D), jnp.float32),
    )(x, degb, wu, bu, wf, bf, w1)


def _mid_body(acc_ref, degb_ref, b_ref, w_ref, out_ref):
    dinv = _dinv_of(degb_ref[...])
    agg = acc_ref[0] + acc_ref[1]
    h = jnp.maximum(dinv * agg + b_ref[...], 0.0)
    m = jnp.dot(h, w_ref[...], preferred_element_type=jnp.float32)
    out_ref[...] = dinv * m


def _mid_call(acc, degb, b, w):
    return pl.pallas_call(
        _mid_body,
        grid=(_GRID,),
        in_specs=[
            pl.BlockSpec((NC, _R, D), lambda i: (0, i, 0)),
            pl.BlockSpec((_R, D), lambda i: (i, 0)),
            pl.BlockSpec((1, D), lambda i: (0, 0)),
            pl.BlockSpec((D, D), lambda i: (0, 0)),
        ],
        out_specs=pl.BlockSpec((_R, D), lambda i: (i, 0)),
        out_shape=jax.ShapeDtypeStruct((N_PAD, D), jnp.float32),
    )(acc, degb, b, w)


def _fin_body(acc_ref, degb_ref, b_ref, out_ref, out2_ref):
    dinv = _dinv_of(degb_ref[...])
    agg = acc_ref[0] + acc_ref[1]
    h = jnp.maximum(dinv * agg + b_ref[...], 0.0)
    out_ref[...] = h
    out2_ref[...] = h


_FR = 2000


def _fin_call(acc, degb, b, n_rows, row0):
    blk0 = row0 // _FR
    return pl.pallas_call(
        _fin_body,
        grid=(n_rows // _FR,),
        in_specs=[
            pl.BlockSpec((NC, _FR, D), lambda i: (0, blk0 + i, 0)),
            pl.BlockSpec((_FR, D), lambda i: (blk0 + i, 0)),
            pl.BlockSpec((1, D), lambda i: (0, 0)),
        ],
        out_specs=[
            pl.BlockSpec((_FR, D), lambda i: (i, 0)),
            pl.BlockSpec((_FR, D), lambda i: (i, 0)),
        ],
        out_shape=[
            jax.ShapeDtypeStruct((n_rows, D), jnp.float32),
            jax.ShapeDtypeStruct((n_rows, D), jnp.float32),
        ],
    )(acc, degb, b)


def kernel(user_x, food_x, edge_index, W_user, b_user, W_food, b_food,
           W1, b1, W2, b2):
    src, dst = lax.optimization_barrier((edge_index[0], edge_index[1]))
    pad_idx = jnp.asarray(
        np.asarray(N_NODES + np.arange(E_PAD - E) % (N_PAD - N_NODES),
                   dtype=np.int32))
    dst_r = jnp.concatenate([dst, pad_idx]).reshape(NW, NCHUNK, CHUNK)
    src_r = jnp.concatenate([src, pad_idx]).reshape(NW, NCHUNK, CHUNK)
    x_pad = jnp.concatenate(
        [user_x, food_x, jnp.zeros((N_PAD - N_NODES, D), jnp.float32)], axis=0)
    zeros1 = jnp.asarray(np.zeros((N_PAD,), np.float32))
    zeros2 = jnp.asarray(np.zeros((N_PAD, D), np.float32))

    degp = _deg_call()(dst_r, zeros1)
    degb = _degb_call(degp)

    m1 = _proj_call(x_pad, degb, W_user, b_user.reshape(1, D),
                    W_food, b_food.reshape(1, D), W1)
    acc1 = _scat_call()(m1, src_r, dst_r, zeros2)
    m2 = _mid_call(acc1, degb, b1.reshape(1, D), W2)
    acc2 = _scat_call()(m2, src_r, dst_r, zeros2)
    users, users2 = _fin_call(acc2, degb, b2.reshape(1, D), N_USERS, 0)
    items, items2 = _fin_call(acc2, degb, b2.reshape(1, D), N_ITEMS, N_USERS)
    return (users, users2, items, items2)


# speedup vs baseline: 26.4904x; 1.0186x over previous
import functools

import jax
import jax.numpy as jnp
import numpy as np
from jax import lax
from jax.experimental import pallas as pl
from jax.experimental.pallas import tpu as pltpu
from jax.experimental.pallas import tpu_sc as plsc

N_USERS = 2000
N_ITEMS = 8000
N_NODES = N_USERS + N_ITEMS
D = 128
E = 320000

NC = 2
NS = 16
NW = NC * NS
CHUNK = 128
NCHUNK = 80
E_PAD = NW * NCHUNK * CHUNK
N_PAD = 10240
RPT = N_PAD // NS

_MESH = dict(core_axis_name="c", subcore_axis_name="s")


def _deg_body(dstr_hbm, zeros1_hbm, out_hbm, dst_v, ones_v, acc_sh, dsem):
    c = lax.axis_index("c")
    s = lax.axis_index("s")
    w = c * NS + s
    pltpu.sync_copy(zeros1_hbm.at[pl.ds(s * RPT, RPT)],
                    acc_sh.at[pl.ds(s * RPT, RPT)])
    pltpu.sync_copy(dstr_hbm.at[w], dst_v)
    for j in range(CHUNK // 16):
        ones_v[pl.ds(j * 16, 16)] = jnp.full((16,), 1.0, dtype=jnp.float32)
    plsc.subcore_barrier()

    WAVE = 16

    def body(iw, carry):
        for k in range(WAVE):
            pltpu.async_copy(ones_v, acc_sh.at[dst_v.at[iw * WAVE + k]],
                             dsem, add=True)
        for k in range(WAVE):
            pltpu.make_async_copy(ones_v, acc_sh.at[dst_v.at[iw * WAVE + k]],
                                  dsem).wait()
        return carry

    lax.fori_loop(0, NCHUNK // WAVE, body, 0)
    plsc.subcore_barrier()
    pltpu.sync_copy(acc_sh.at[pl.ds(s * RPT, RPT)],
                    out_hbm.at[c, pl.ds(s * RPT, RPT)])


@functools.cache
def _deg_call():
    return pl.kernel(
        _deg_body,
        out_type=jax.ShapeDtypeStruct((NC, N_PAD), jnp.float32),
        mesh=plsc.VectorSubcoreMesh(**_MESH),
        scratch_types=[
            pltpu.VMEM((NCHUNK, CHUNK), jnp.int32),
            pltpu.VMEM((CHUNK,), jnp.float32),
            pltpu.VMEM_SHARED((N_PAD,), jnp.float32),
            pltpu.SemaphoreType.DMA,
        ],
    )


GROUP = 16
NGROUP = NCHUNK // GROUP


def _scat_body(table_hbm, srcr_hbm, dstr_hbm, zeros2_hbm, out_hbm,
               srcs0, srcs1, dsts0, dsts1, rows0, rows1, acc_sh,
               i0sem, i1sem, g0, g1, s0, s1, zsem):
    c = lax.axis_index("c")
    s = lax.axis_index("s")
    w = c * NS + s

    src_s = (srcs0, srcs1)
    dst_s = (dsts0, dsts1)
    isem = (i0sem, i1sem)
    bufs = (rows0, rows1)
    gsem = (g0, g1)
    ssem = (s0, s1)

    def idx_start(g, slot):
        pltpu.async_copy(srcr_hbm.at[w, pl.ds(g * GROUP, GROUP)],
                         src_s[slot], isem[slot])
        pltpu.async_copy(dstr_hbm.at[w, pl.ds(g * GROUP, GROUP)],
                         dst_s[slot], isem[slot])

    def idx_wait(g, slot):
        pltpu.make_async_copy(srcr_hbm.at[w, pl.ds(g * GROUP, GROUP)],
                              src_s[slot], isem[slot]).wait()
        pltpu.make_async_copy(dstr_hbm.at[w, pl.ds(g * GROUP, GROUP)],
                              dst_s[slot], isem[slot]).wait()

    def g_start(slot, j, b):
        pltpu.async_copy(table_hbm.at[src_s[slot].at[j]], bufs[b], gsem[b])

    def g_wait(slot, j, b):
        pltpu.make_async_copy(table_hbm.at[src_s[slot].at[j]], bufs[b],
                              gsem[b]).wait()

    def s_start(slot, j, b):
        pltpu.async_copy(bufs[b], acc_sh.at[dst_s[slot].at[j]], ssem[b],
                         add=True)

    def s_wait(slot, j, b):
        pltpu.make_async_copy(bufs[b], acc_sh.at[dst_s[slot].at[j]],
                              ssem[b]).wait()

    pltpu.async_copy(zeros2_hbm.at[pl.ds(s * RPT, RPT)],
                     acc_sh.at[pl.ds(s * RPT, RPT)], zsem)
    idx_start(0, 0)
    idx_wait(0, 0)
    g_start(0, 0, 0)
    g_start(0, 1, 1)
    pltpu.make_async_copy(zeros2_hbm.at[pl.ds(s * RPT, RPT)],
                          acc_sh.at[pl.ds(s * RPT, RPT)], zsem).wait()
    plsc.subcore_barrier()

    for g in range(NGROUP):
        slot = g % 2
        if g + 1 < NGROUP:
            idx_start(g + 1, 1 - slot)

        def ibody(it, carry, slot=slot):
            for k in (0, 1):
                j = it * 2 + k
                g_wait(slot, j, k)
                s_start(slot, j, k)
                s_wait(slot, j, k)
                g_start(slot, j + 2, k)
            return carry

        lax.fori_loop(0, GROUP // 2 - 1, ibody, 0)
        for j in (GROUP - 2, GROUP - 1):
            b = j % 2
            g_wait(slot, j, b)
            s_start(slot, j, b)
            s_wait(slot, j, b)
            if g + 1 < NGROUP:
                if j == GROUP - 2:
                    idx_wait(g + 1, 1 - slot)
                g_start(1 - slot, j - (GROUP - 2), b)

    plsc.subcore_barrier()
    pltpu.sync_copy(acc_sh.at[pl.ds(s * RPT, RPT)],
                    out_hbm.at[c, pl.ds(s * RPT, RPT)])


@functools.cache
def _scat_call():
    return pl.kernel(
        _scat_body,
        out_type=jax.ShapeDtypeStruct((NC, N_PAD, D), jnp.float32),
        mesh=plsc.VectorSubcoreMesh(**_MESH),
        scratch_types=[
            pltpu.VMEM((GROUP, CHUNK), jnp.int32),
            pltpu.VMEM((GROUP, CHUNK), jnp.int32),
            pltpu.VMEM((GROUP, CHUNK), jnp.int32),
            pltpu.VMEM((GROUP, CHUNK), jnp.int32),
            pltpu.VMEM((CHUNK, D), jnp.float32),
            pltpu.VMEM((CHUNK, D), jnp.float32),
            pltpu.VMEM_SHARED((N_PAD, D), jnp.float32),
            pltpu.SemaphoreType.DMA,
            pltpu.SemaphoreType.DMA,
            pltpu.SemaphoreType.DMA,
            pltpu.SemaphoreType.DMA,
            pltpu.SemaphoreType.DMA,
            pltpu.SemaphoreType.DMA,
            pltpu.SemaphoreType.DMA,
        ],
    )


_R = 2048
_GRID = N_PAD // _R


def _dinv_of(degp):
    degb = lax.dot_general(
        degp, jnp.ones((NC, D), jnp.float32),
        (((0,), (0,)), ((), ())), preferred_element_type=jnp.float32)
    return jnp.where(degb > 0.0, lax.rsqrt(jnp.maximum(degb, 1.0)), 0.0)


_DEGP_SPEC = pl.BlockSpec((NC, _R), lambda i: (0, i))


def _degb_body(degp_ref, out_ref):
    out_ref[...] = _dinv_of(degp_ref[...])


def _degb_call(degp):
    return pl.pallas_call(
        _degb_body,
        grid=(_GRID,),
        in_specs=[_DEGP_SPEC],
        out_specs=pl.BlockSpec((_R, D), lambda i: (i, 0)),
        out_shape=jax.ShapeDtypeStruct((N_PAD, D), jnp.float32),
    )(degp)


def _proj_body(x_ref, degp_ref, wu_ref, bu_ref, wf_ref, bf_ref, w1_ref,
               out_ref):
    i = pl.program_id(0)
    x = x_ref[...]
    hu = jnp.maximum(jnp.dot(x, wu_ref[...],
                             preferred_element_type=jnp.float32)
                     + bu_ref[...], 0.0)
    hf = jnp.maximum(jnp.dot(x, wf_ref[...],
                             preferred_element_type=jnp.float32)
                     + bf_ref[...], 0.0)
    row = i * _R + lax.broadcasted_iota(jnp.int32, (_R, 1), 0)
    h = jnp.where(row < N_USERS, hu, hf)
    m = jnp.dot(h, w1_ref[...], preferred_element_type=jnp.float32)
    out_ref[...] = _dinv_of(degp_ref[...]) * m


def _proj_call(x, degp, wu, bu, wf, bf, w1):
    return pl.pallas_call(
        _proj_body,
        grid=(_GRID,),
        in_specs=[
            pl.BlockSpec((_R, D), lambda i: (i, 0)),
            _DEGP_SPEC,
            pl.BlockSpec((D, D), lambda i: (0, 0)),
            pl.BlockSpec((1, D), lambda i: (0, 0)),
            pl.BlockSpec((D, D), lambda i: (0, 0)),
            pl.BlockSpec((1, D), lambda i: (0, 0)),
            pl.BlockSpec((D, D), lambda i: (0, 0)),
        ],
        out_specs=pl.BlockSpec((_R, D), lambda i: (i, 0)),
        out_shape=jax.ShapeDtypeStruct((N_PAD, D), jnp.float32),
    )(x, degp, wu, bu, wf, bf, w1)


def _mid_body(acc_ref, degp_ref, b_ref, w_ref, out_ref):
    dinv = _dinv_of(degp_ref[...])
    agg = acc_ref[0] + acc_ref[1]
    h = jnp.maximum(dinv * agg + b_ref[...], 0.0)
    m = jnp.dot(h, w_ref[...], preferred_element_type=jnp.float32)
    out_ref[...] = dinv * m


def _mid_call(acc, degp, b, w):
    return pl.pallas_call(
        _mid_body,
        grid=(_GRID,),
        in_specs=[
            pl.BlockSpec((NC, _R, D), lambda i: (0, i, 0)),
            _DEGP_SPEC,
            pl.BlockSpec((1, D), lambda i: (0, 0)),
            pl.BlockSpec((D, D), lambda i: (0, 0)),
        ],
        out_specs=pl.BlockSpec((_R, D), lambda i: (i, 0)),
        out_shape=jax.ShapeDtypeStruct((N_PAD, D), jnp.float32),
    )(acc, degp, b, w)


def _fin_body(acc_ref, dinvb_ref, b_ref, out_ref, out2_ref):
    dinv = dinvb_ref[...]
    agg = acc_ref[0] + acc_ref[1]
    h = jnp.maximum(dinv * agg + b_ref[...], 0.0)
    out_ref[...] = h
    out2_ref[...] = h


_FR = 2000


def _fin_call(acc, dinvb, b, n_rows, row0):
    blk0 = row0 // _FR
    return pl.pallas_call(
        _fin_body,
        grid=(n_rows // _FR,),
        in_specs=[
            pl.BlockSpec((NC, _FR, D), lambda i: (0, blk0 + i, 0)),
            pl.BlockSpec((_FR, D), lambda i: (blk0 + i, 0)),
            pl.BlockSpec((1, D), lambda i: (0, 0)),
        ],
        out_specs=[
            pl.BlockSpec((_FR, D), lambda i: (i, 0)),
            pl.BlockSpec((_FR, D), lambda i: (i, 0)),
        ],
        out_shape=[
            jax.ShapeDtypeStruct((n_rows, D), jnp.float32),
            jax.ShapeDtypeStruct((n_rows, D), jnp.float32),
        ],
    )(acc, dinvb, b)


def kernel(user_x, food_x, edge_index, W_user, b_user, W_food, b_food,
           W1, b1, W2, b2):
    src, dst = lax.optimization_barrier((edge_index[0], edge_index[1]))
    pad_idx = jnp.asarray(
        np.asarray(N_NODES + np.arange(E_PAD - E) % (N_PAD - N_NODES),
                   dtype=np.int32))
    dst_r = jnp.concatenate([dst, pad_idx]).reshape(NW, NCHUNK, CHUNK)
    src_r = jnp.concatenate([src, pad_idx]).reshape(NW, NCHUNK, CHUNK)
    x_pad = jnp.concatenate(
        [user_x, food_x, jnp.zeros((N_PAD - N_NODES, D), jnp.float32)], axis=0)
    zeros1 = jnp.asarray(np.zeros((N_PAD,), np.float32))
    z0 = (edge_index[0, 0] * 0).astype(jnp.float32)
    zeros2 = jnp.broadcast_to(z0, (N_PAD, D))

    degp = _deg_call()(dst_r, zeros1)

    m1 = _proj_call(x_pad, degp, W_user, b_user.reshape(1, D),
                    W_food, b_food.reshape(1, D), W1)
    acc1 = _scat_call()(m1, src_r, dst_r, zeros2)
    m2 = _mid_call(acc1, degp, b1.reshape(1, D), W2)
    dinvb = _degb_call(degp)
    acc2 = _scat_call()(m2, src_r, dst_r, zeros2)
    users, users2 = _fin_call(acc2, dinvb, b2.reshape(1, D), N_USERS, 0)
    items, items2 = _fin_call(acc2, dinvb, b2.reshape(1, D), N_ITEMS, N_USERS)
    return (users, users2, items, items2)


# speedup vs baseline: 27.1138x; 1.0235x over previous
import functools

import jax
import jax.numpy as jnp
import numpy as np
from jax import lax
from jax.experimental import pallas as pl
from jax.experimental.pallas import tpu as pltpu
from jax.experimental.pallas import tpu_sc as plsc

N_USERS = 2000
N_ITEMS = 8000
N_NODES = N_USERS + N_ITEMS
D = 128
E = 320000

NC = 2
NS = 16
NW = NC * NS
CHUNK = 128
NCHT = E // CHUNK
NCHB = NCHT // NW
NXTRA = NCHT - NCHB * NW
N_PAD = 10240
RPT = N_PAD // NS

_MESH = dict(core_axis_name="c", subcore_axis_name="s")


def _chunk0(w):
    return w * NCHB + jnp.minimum(w, NXTRA)


def _repack_rows(ei2, idx2d, n_full, base=0):
    def rbody(r, carry):
        for l in range(8):
            idx2d[r, pl.ds(l * 16, 16)] = ei2[1, pl.ds(base + r * CHUNK
                                                       + l * 16, 16)]
        return carry

    lax.fori_loop(0, n_full, rbody, 0)


def _deg_body(ei_hbm, zeros1_hbm, out_hbm, ei_v, dst2d, ones_v, acc_sh,
              dsem):
    c = lax.axis_index("c")
    s = lax.axis_index("s")
    w = c * NS + s
    c0 = _chunk0(w)
    pltpu.sync_copy(zeros1_hbm.at[pl.ds(s * RPT, RPT)],
                    acc_sh.at[pl.ds(s * RPT, RPT)])
    pltpu.sync_copy(ei_hbm.at[:, pl.ds(c0 * CHUNK, NCHB * CHUNK)],
                    ei_v.at[:, pl.ds(0, NCHB * CHUNK)])

    @pl.when(w < NXTRA)
    def _stage_extra():
        pltpu.sync_copy(ei_hbm.at[:, pl.ds((c0 + NCHB) * CHUNK, CHUNK)],
                        ei_v.at[:, pl.ds(NCHB * CHUNK, CHUNK)])

    for j in range(CHUNK // 16):
        ones_v[pl.ds(j * 16, 16)] = jnp.full((16,), 1.0, dtype=jnp.float32)
    _repack_rows(ei_v, dst2d, NCHB)

    @pl.when(w < NXTRA)
    def _repack_extra():
        for l in range(8):
            dst2d[NCHB, pl.ds(l * 16, 16)] = ei_v[1, pl.ds(NCHB * CHUNK
                                                           + l * 16, 16)]

    plsc.subcore_barrier()

    WAVE = 16

    def body(iw, carry):
        for k in range(WAVE):
            pltpu.async_copy(ones_v, acc_sh.at[dst2d.at[iw * WAVE + k]],
                             dsem, add=True)
        for k in range(WAVE):
            pltpu.make_async_copy(ones_v, acc_sh.at[dst2d.at[iw * WAVE + k]],
                                  dsem).wait()
        return carry

    lax.fori_loop(0, NCHB // WAVE, body, 0)
    for i in range(NCHB - NCHB % WAVE, NCHB):
        pltpu.async_copy(ones_v, acc_sh.at[dst2d.at[i]], dsem, add=True)
    for i in range(NCHB - NCHB % WAVE, NCHB):
        pltpu.make_async_copy(ones_v, acc_sh.at[dst2d.at[i]], dsem).wait()

    @pl.when(w < NXTRA)
    def _scat_extra():
        pltpu.sync_copy(ones_v, acc_sh.at[dst2d.at[NCHB]], add=True)

    plsc.subcore_barrier()
    pltpu.sync_copy(acc_sh.at[pl.ds(s * RPT, RPT)],
                    out_hbm.at[c, pl.ds(s * RPT, RPT)])


@functools.cache
def _deg_call():
    return pl.kernel(
        _deg_body,
        out_type=jax.ShapeDtypeStruct((NC, N_PAD), jnp.float32),
        mesh=plsc.VectorSubcoreMesh(**_MESH),
        scratch_types=[
            pltpu.VMEM((2, (NCHB + 1) * CHUNK), jnp.int32),
            pltpu.VMEM((NCHB + 1, CHUNK), jnp.int32),
            pltpu.VMEM((CHUNK,), jnp.float32),
            pltpu.VMEM_SHARED((N_PAD,), jnp.float32),
            pltpu.SemaphoreType.DMA,
        ],
    )


GROUP = 16
NGROUP = 5
_SZ = (2048, 2048, 2048, 2048, 1792)
_CPG = (16, 16, 16, 16, 14)


def _scat_body(table_hbm, ei_hbm, zeros2_hbm, out_hbm,
               eis0, eis1, dst2d0, dst2d1, rows0, rows1,
               acc_sh, i0sem, i1sem, g0, g1, s0, s1, zsem):
    c = lax.axis_index("c")
    s = lax.axis_index("s")
    w = c * NS + s
    c0 = _chunk0(w)

    ei_s = (eis0, eis1)
    dst_s = (dst2d0, dst2d1)
    isem = (i0sem, i1sem)
    bufs = (rows0, rows1)
    gsem = (g0, g1)
    ssem = (s0, s1)

    def idx_start(g, slot):
        off = (c0 + g * GROUP) * CHUNK
        sz = _SZ[g]
        pltpu.async_copy(ei_hbm.at[:, pl.ds(off, sz)],
                         ei_s[slot].at[:, pl.ds(0, sz)], isem[slot])

    def idx_wait(g, slot):
        off = (c0 + g * GROUP) * CHUNK
        sz = _SZ[g]
        pltpu.make_async_copy(ei_hbm.at[:, pl.ds(off, sz)],
                              ei_s[slot].at[:, pl.ds(0, sz)],
                              isem[slot]).wait()

    def repack(g, slot):
        _repack_rows(ei_s[slot], dst_s[slot], _CPG[g])

    def g_start(slot, j, b):
        pltpu.async_copy(
            table_hbm.at[ei_s[slot].at[0, pl.ds(j * CHUNK, CHUNK)]],
            bufs[b], gsem[b])

    def g_wait(slot, j, b):
        pltpu.make_async_copy(
            table_hbm.at[ei_s[slot].at[0, pl.ds(j * CHUNK, CHUNK)]],
            bufs[b], gsem[b]).wait()

    def s_start(slot, j, b):
        pltpu.async_copy(bufs[b], acc_sh.at[dst_s[slot].at[j]], ssem[b],
                         add=True)

    def s_wait(slot, j, b):
        pltpu.make_async_copy(bufs[b], acc_sh.at[dst_s[slot].at[j]],
                              ssem[b]).wait()

    pltpu.async_copy(zeros2_hbm.at[pl.ds(s * RPT, RPT)],
                     acc_sh.at[pl.ds(s * RPT, RPT)], zsem)
    idx_start(0, 0)
    idx_wait(0, 0)
    g_start(0, 0, 0)
    g_start(0, 1, 1)
    repack(0, 0)
    pltpu.make_async_copy(zeros2_hbm.at[pl.ds(s * RPT, RPT)],
                          acc_sh.at[pl.ds(s * RPT, RPT)], zsem).wait()
    plsc.subcore_barrier()

    for g in range(NGROUP - 1):
        slot = g % 2
        idx_start(g + 1, 1 - slot)

        def ibody(it, carry, slot=slot):
            for k in (0, 1):
                j = it * 2 + k
                g_wait(slot, j, k)
                s_start(slot, j, k)
                s_wait(slot, j, k)
                g_start(slot, j + 2, k)
            return carry

        lax.fori_loop(0, GROUP // 2 - 1, ibody, 0)
        for j in (GROUP - 2, GROUP - 1):
            b = j % 2
            g_wait(slot, j, b)
            s_start(slot, j, b)
            s_wait(slot, j, b)
            if j == GROUP - 2:
                idx_wait(g + 1, 1 - slot)
                repack(g + 1, 1 - slot)
            g_start(1 - slot, j - (GROUP - 2), b)

    gl = NGROUP - 1
    slot = gl % 2

    def lbody(it, carry, slot=slot):
        for k in (0, 1):
            j = it * 2 + k
            g_wait(slot, j, k)
            s_start(slot, j, k)
            s_wait(slot, j, k)
            g_start(slot, j + 2, k)
        return carry

    lax.fori_loop(0, (_CPG[gl] - 2) // 2, lbody, 0)
    for j in (_CPG[gl] - 2, _CPG[gl] - 1):
        g_wait(slot, j, j % 2)
        s_start(slot, j, j % 2)
        s_wait(slot, j, j % 2)

    @pl.when(w < NXTRA)
    def _extra():
        off = (c0 + NCHB) * CHUNK
        pltpu.sync_copy(ei_hbm.at[:, pl.ds(off, CHUNK)],
                        ei_s[1].at[:, pl.ds(0, CHUNK)])
        for l in range(8):
            dst_s[1][0, pl.ds(l * 16, 16)] = ei_s[1][1, pl.ds(l * 16, 16)]
        pltpu.async_copy(table_hbm.at[ei_s[1].at[0, pl.ds(0, CHUNK)]],
                         bufs[0], gsem[0])
        pltpu.make_async_copy(table_hbm.at[ei_s[1].at[0, pl.ds(0, CHUNK)]],
                              bufs[0], gsem[0]).wait()
        pltpu.sync_copy(bufs[0], acc_sh.at[dst_s[1].at[0]], add=True)

    plsc.subcore_barrier()
    pltpu.sync_copy(acc_sh.at[pl.ds(s * RPT, RPT)],
                    out_hbm.at[c, pl.ds(s * RPT, RPT)])


@functools.cache
def _scat_call():
    return pl.kernel(
        _scat_body,
        out_type=jax.ShapeDtypeStruct((NC, N_PAD, D), jnp.float32),
        mesh=plsc.VectorSubcoreMesh(**_MESH),
        scratch_types=[
            pltpu.VMEM((2, GROUP * CHUNK), jnp.int32),
            pltpu.VMEM((2, GROUP * CHUNK), jnp.int32),
            pltpu.VMEM((GROUP, CHUNK), jnp.int32),
            pltpu.VMEM((GROUP, CHUNK), jnp.int32),
            pltpu.VMEM((CHUNK, D), jnp.float32),
            pltpu.VMEM((CHUNK, D), jnp.float32),
            pltpu.VMEM_SHARED((N_PAD, D), jnp.float32),
            pltpu.SemaphoreType.DMA,
            pltpu.SemaphoreType.DMA,
            pltpu.SemaphoreType.DMA,
            pltpu.SemaphoreType.DMA,
            pltpu.SemaphoreType.DMA,
            pltpu.SemaphoreType.DMA,
            pltpu.SemaphoreType.DMA,
        ],
    )


_R = 2048
_GRID = N_PAD // _R


def _dinv_of(degp):
    degb = lax.dot_general(
        degp, jnp.ones((NC, D), jnp.float32),
        (((0,), (0,)), ((), ())), preferred_element_type=jnp.float32)
    return jnp.where(degb > 0.0, lax.rsqrt(jnp.maximum(degb, 1.0)), 0.0)


_DEGP_SPEC = pl.BlockSpec((NC, _R), lambda i: (0, i))


def _degb_body(degp_ref, out_ref):
    out_ref[...] = _dinv_of(degp_ref[...])


def _degb_call(degp):
    return pl.pallas_call(
        _degb_body,
        grid=(_GRID,),
        in_specs=[_DEGP_SPEC],
        out_specs=pl.BlockSpec((_R, D), lambda i: (i, 0)),
        out_shape=jax.ShapeDtypeStruct((N_PAD, D), jnp.float32),
    )(degp)


def _proj_body(x_ref, degp_ref, wu_ref, bu_ref, wf_ref, bf_ref, w1_ref,
               out_ref):
    i = pl.program_id(0)
    x = x_ref[...]
    hu = jnp.maximum(jnp.dot(x, wu_ref[...],
                             preferred_element_type=jnp.float32)
                     + bu_ref[...], 0.0)
    hf = jnp.maximum(jnp.dot(x, wf_ref[...],
                             preferred_element_type=jnp.float32)
                     + bf_ref[...], 0.0)
    row = i * _R + lax.broadcasted_iota(jnp.int32, (_R, 1), 0)
    h = jnp.where(row < N_USERS, hu, hf)
    m = jnp.dot(h, w1_ref[...], preferred_element_type=jnp.float32)
    out_ref[...] = _dinv_of(degp_ref[...]) * m


def _proj_call(x, degp, wu, bu, wf, bf, w1):
    return pl.pallas_call(
        _proj_body,
        grid=(_GRID,),
        in_specs=[
            pl.BlockSpec((_R, D), lambda i: (i, 0)),
            _DEGP_SPEC,
            pl.BlockSpec((D, D), lambda i: (0, 0)),
            pl.BlockSpec((1, D), lambda i: (0, 0)),
            pl.BlockSpec((D, D), lambda i: (0, 0)),
            pl.BlockSpec((1, D), lambda i: (0, 0)),
            pl.BlockSpec((D, D), lambda i: (0, 0)),
        ],
        out_specs=pl.BlockSpec((_R, D), lambda i: (i, 0)),
        out_shape=jax.ShapeDtypeStruct((N_PAD, D), jnp.float32),
    )(x, degp, wu, bu, wf, bf, w1)


def _mid_body(acc_ref, degp_ref, b_ref, w_ref, out_ref):
    dinv = _dinv_of(degp_ref[...])
    agg = acc_ref[0] + acc_ref[1]
    h = jnp.maximum(dinv * agg + b_ref[...], 0.0)
    m = jnp.dot(h, w_ref[...], preferred_element_type=jnp.float32)
    out_ref[...] = dinv * m


def _mid_call(acc, degp, b, w):
    return pl.pallas_call(
        _mid_body,
        grid=(_GRID,),
        in_specs=[
            pl.BlockSpec((NC, _R, D), lambda i: (0, i, 0)),
            _DEGP_SPEC,
            pl.BlockSpec((1, D), lambda i: (0, 0)),
            pl.BlockSpec((D, D), lambda i: (0, 0)),
        ],
        out_specs=pl.BlockSpec((_R, D), lambda i: (i, 0)),
        out_shape=jax.ShapeDtypeStruct((N_PAD, D), jnp.float32),
    )(acc, degp, b, w)


def _fin_body(acc_ref, dinvb_ref, b_ref, out_ref, out2_ref):
    dinv = dinvb_ref[...]
    agg = acc_ref[0] + acc_ref[1]
    h = jnp.maximum(dinv * agg + b_ref[...], 0.0)
    out_ref[...] = h
    out2_ref[...] = h


_FR = 2000


def _fin_call(acc, dinvb, b, n_rows, row0):
    blk0 = row0 // _FR
    return pl.pallas_call(
        _fin_body,
        grid=(n_rows // _FR,),
        in_specs=[
            pl.BlockSpec((NC, _FR, D), lambda i: (0, blk0 + i, 0)),
            pl.BlockSpec((_FR, D), lambda i: (blk0 + i, 0)),
            pl.BlockSpec((1, D), lambda i: (0, 0)),
        ],
        out_specs=[
            pl.BlockSpec((_FR, D), lambda i: (i, 0)),
            pl.BlockSpec((_FR, D), lambda i: (i, 0)),
        ],
        out_shape=[
            jax.ShapeDtypeStruct((n_rows, D), jnp.float32),
            jax.ShapeDtypeStruct((n_rows, D), jnp.float32),
        ],
    )(acc, dinvb, b)


def kernel(user_x, food_x, edge_index, W_user, b_user, W_food, b_food,
           W1, b1, W2, b2):
    x_pad = jnp.concatenate(
        [user_x, food_x, jnp.zeros((N_PAD - N_NODES, D), jnp.float32)], axis=0)
    zeros1 = jnp.asarray(np.zeros((N_PAD,), np.float32))
    z0 = (edge_index[0, 0] * 0).astype(jnp.float32)
    zeros2 = jnp.broadcast_to(z0, (N_PAD, D))

    degp = _deg_call()(edge_index, zeros1)

    m1 = _proj_call(x_pad, degp, W_user, b_user.reshape(1, D),
                    W_food, b_food.reshape(1, D), W1)
    acc1 = _scat_call()(m1, edge_index, zeros2)
    m2 = _mid_call(acc1, degp, b1.reshape(1, D), W2)
    dinvb = _degb_call(degp)
    acc2 = _scat_call()(m2, edge_index, zeros2)
    users, users2 = _fin_call(acc2, dinvb, b2.reshape(1, D), N_USERS, 0)
    items, items2 = _fin_call(acc2, dinvb, b2.reshape(1, D), N_ITEMS, N_USERS)
    return (users, users2, items, items2)
